# Initial kernel scaffold; baseline (speedup 1.0000x reference)
import jax
import jax.numpy as jnp
from jax.experimental import pallas as pl


def kernel(x, edge_index, edge_attr, W1a, b1a, W1b, b1b, W1c, b1c, W1d, b1d, W2a, b2a, W2b, b2b, W2c, b2c, W2d, b2d):
    raise NotImplementedError("write your pallas kernel here")


# speedup vs baseline: 7.5667x; 7.5667x over previous
import functools

import jax
import jax.numpy as jnp
from jax import lax
from jax.experimental import pallas as pl
from jax.experimental.pallas import tpu as pltpu
from jax.experimental.pallas import tpu_sc as plsc

N = 50000
E = 800000
F_IN = 128
H = 64
OUT = 2
HQ = 32
NQ = H // HQ

NC = 2
NS = 16
LANES = 128
NCH = 392
G = 14
NG = NCH // G
EP = NS * NCH * LANES
EROWS = EP // LANES
TRASH = N
ACC_ROWS = 51200
STRIPE = ACC_ROWS // NS
ZROWS = 64

BM = 1024

_f32 = jnp.float32


@functools.lru_cache(maxsize=None)
def _make_sc_pass(with_cnt: bool):
    mesh = plsc.VectorSubcoreMesh(core_axis_name="c", subcore_axis_name="s",
                                  num_cores=NC, num_subcores=NS)
    out_type = [jax.ShapeDtypeStruct((NC, ACC_ROWS, HQ), _f32)]
    scratch = [
        pltpu.VMEM((2, G, LANES), jnp.int32),
        pltpu.VMEM((2, G, LANES), jnp.int32),
        pltpu.VMEM((2, LANES, HQ), _f32),
        pltpu.VMEM((ZROWS, HQ), _f32),
        pltpu.MemorySpace.VMEM_SHARED((ACC_ROWS, HQ), _f32),
        pltpu.SemaphoreType.DMA,
        pltpu.SemaphoreType.DMA,
        pltpu.SemaphoreType.DMA,
    ]
    if with_cnt:
        out_type.append(jax.ShapeDtypeStruct((NC, ACC_ROWS), _f32))
        scratch += [
            pltpu.VMEM((LANES,), _f32),
            pltpu.VMEM((LANES,), _f32),
            pltpu.MemorySpace.VMEM_SHARED((ACC_ROWS,), _f32),
        ]

    def body(m_cat, rows_a, rows_b, cols, *refs):
        if with_cnt:
            (s_out, cnt_out, idxr_v, idxc_v, val_v, zbuf, acc_sh, sem0, sem1,
             isem, ones_v, zrow, cnt_sh) = refs
        else:
            (s_out, idxr_v, idxc_v, val_v, zbuf, acc_sh, sem0, sem1, isem) = refs
        c = lax.axis_index("c")
        s = lax.axis_index("s")
        base = s * NCH
        stripe0 = s * STRIPE

        def _stage(g, slot):
            @pl.when(c == 0)
            def _():
                pltpu.async_copy(rows_a.at[pl.ds(base + g * G, G)],
                                 idxr_v.at[slot], isem)

            @pl.when(c == 1)
            def _():
                pltpu.async_copy(rows_b.at[pl.ds(base + g * G, G)],
                                 idxr_v.at[slot], isem)

            pltpu.async_copy(cols.at[pl.ds(base + g * G, G)],
                             idxc_v.at[slot], isem)

        def _stage_wait(slot):
            pltpu.make_async_copy(rows_a.at[pl.ds(base, G)], idxr_v.at[slot],
                                  isem).wait()
            pltpu.make_async_copy(cols.at[pl.ds(base, G)], idxc_v.at[slot],
                                  isem).wait()

        def _zfill(i, carry):
            zbuf[i, pl.ds(0, 16)] = jnp.zeros((16,), _f32)
            zbuf[i, pl.ds(16, 16)] = jnp.zeros((16,), _f32)
            return carry

        lax.fori_loop(0, ZROWS, _zfill, 0)
        if with_cnt:
            for k in range(LANES // 16):
                ones_v[pl.ds(k * 16, 16)] = jnp.ones((16,), _f32)
                zrow[pl.ds(k * 16, 16)] = jnp.zeros((16,), _f32)

        _stage(0, 0)

        def _zacc(j, carry):
            pltpu.sync_copy(zbuf, acc_sh.at[pl.ds(stripe0 + j * ZROWS, ZROWS)])
            return carry

        lax.fori_loop(0, STRIPE // ZROWS, _zacc, 0)
        if with_cnt:
            def _zcnt(j, carry):
                pltpu.sync_copy(zrow, cnt_sh.at[pl.ds(stripe0 + j * LANES, LANES)])
                return carry

            lax.fori_loop(0, STRIPE // LANES, _zcnt, 0)
        plsc.subcore_barrier()

        def _start(slot, j, buf, sem):
            pltpu.async_copy(m_cat.at[idxr_v.at[slot, j]], val_v.at[buf], sem)

        def _wait(buf, sem):
            pltpu.make_async_copy(m_cat.at[idxr_v.at[0, 0]], val_v.at[buf],
                                  sem).wait()

        def _scatter(slot, j, buf):
            pltpu.sync_copy(val_v.at[buf], acc_sh.at[idxc_v.at[slot, j]],
                            add=True)

        def _count(g, slot, j):
            do = ((c == 0) & (g < NG // 2)) | ((c == 1) & (g >= NG // 2))

            @pl.when(do)
            def _():
                pltpu.sync_copy(ones_v, cnt_sh.at[idxc_v.at[slot, j]], add=True)

        def _group(g, carry):
            slot = jnp.bitwise_and(g, 1)
            _stage_wait(slot)

            @pl.when(g + 1 < NG)
            def _():
                _stage(g + 1, 1 - slot)

            _start(slot, 0, 0, sem0)
            for j in range(0, G, 2):
                if j + 1 < G:
                    _start(slot, j + 1, 1, sem1)
                _wait(0, sem0)
                _scatter(slot, j, 0)
                if with_cnt:
                    _count(g, slot, j)
                if j + 2 < G:
                    _start(slot, j + 2, 0, sem0)
                if j + 1 < G:
                    _wait(1, sem1)
                    _scatter(slot, j + 1, 1)
                    if with_cnt:
                        _count(g, slot, j + 1)
            return carry

        lax.fori_loop(0, NG, _group, 0)
        plsc.subcore_barrier()

        pltpu.sync_copy(acc_sh.at[pl.ds(stripe0, STRIPE)],
                        s_out.at[c, pl.ds(stripe0, STRIPE)])
        if with_cnt:
            pltpu.sync_copy(cnt_sh.at[pl.ds(stripe0, STRIPE)],
                            cnt_out.at[c, pl.ds(stripe0, STRIPE)])

    return pl.kernel(
        body, out_type=out_type, mesh=mesh, scratch_types=scratch,
        compiler_params=pltpu.CompilerParams(use_tc_tiling_on_sc=False))


def _dot(a, b):
    return jnp.dot(a, b, preferred_element_type=_f32)


def _t1_body(x_ref, wa, ba, wb, bb, o_ref):
    h = jnp.maximum(_dot(x_ref[...], wa[...]) + ba[...], 0.0)
    o_ref[...] = _dot(h, wb[...]) + bb[...]


def _t2_body(x_ref, s0, s1, c0, c1, w1c0, w1c1, b1c, w1d, b1d,
             w2a0, w2a1, b2a, w2b, b2b, u_ref, m2_ref):
    cnt = jnp.maximum(c0[...] + c1[...], 1.0)
    agg = jnp.concatenate([s0[...], s1[...]], axis=1) / cnt
    t = _dot(x_ref[...], w1c0[...]) + _dot(agg, w1c1[...]) + b1c[...]
    u = _dot(jnp.maximum(t, 0.0), w1d[...]) + b1d[...]
    u_ref[...] = u
    t2 = _dot(x_ref[...], w2a0[...]) + _dot(u, w2a1[...]) + b2a[...]
    m2_ref[...] = _dot(jnp.maximum(t2, 0.0), w2b[...]) + b2b[...]


def _t3_body(x_ref, u, s0, s1, c0, c1, w2c0, w2c1, w2c2, b2c, wd, bd,
             o_ref):
    cnt = jnp.maximum(c0[...] + c1[...], 1.0)
    agg = jnp.concatenate([s0[...], s1[...]], axis=1) / cnt
    t = (_dot(x_ref[...], w2c0[...]) + _dot(u[...], w2c1[...])
         + _dot(agg, w2c2[...]) + b2c[...])
    o_ref[...] = _dot(jnp.maximum(t, 0.0), wd[...]) + bd[...]


def _row_spec(width):
    return pl.BlockSpec((BM, width), lambda i: (i, 0))


def _full_spec(shape):
    return pl.BlockSpec(shape, lambda i: (0,) * len(shape))


def _tc_call(body, in_widths, full_shapes, out_widths):
    grid = (pl.cdiv(N, BM),)
    out_shape = [jax.ShapeDtypeStruct((N, w), _f32) for w in out_widths]
    out_specs = [_row_spec(w) for w in out_widths]
    if len(out_widths) == 1:
        out_shape, out_specs = out_shape[0], out_specs[0]
    return pl.pallas_call(
        body,
        grid=grid,
        in_specs=[_row_spec(w) for w in in_widths] + [_full_spec(s) for s in full_shapes],
        out_specs=out_specs,
        out_shape=out_shape,
    )


def _stack_halves(m):
    return jnp.concatenate([m[:, q * HQ:(q + 1) * HQ] for q in range(NQ)], axis=0)


def _edge_scatter(m, rows_q, cols_p, first):
    m_cat = _stack_halves(m)
    if first:
        sa, cnt = _make_sc_pass(True)(m_cat, rows_q[0], rows_q[1], cols_p)
        return ((sa[0, :N], sa[1, :N]),
                (cnt[0, :N].reshape(N, 1), cnt[1, :N].reshape(N, 1)))
    (sa,) = _make_sc_pass(False)(m_cat, rows_q[0], rows_q[1], cols_p)
    return (sa[0, :N], sa[1, :N]), None


def kernel(x, edge_index, edge_attr, W1a, b1a, W1b, b1b, W1c, b1c, W1d, b1d,
           W2a, b2a, W2b, b2b, W2c, b2c, W2d, b2d):
    del edge_attr
    row = edge_index[0].astype(jnp.int32)
    col = edge_index[1].astype(jnp.int32)
    pad = EP - E
    row_p = jnp.concatenate([row, jnp.zeros((pad,), jnp.int32)]).reshape(EROWS, LANES)
    rows_q = [row_p + q * N for q in range(NQ)]
    cols_p = jnp.concatenate([col, jnp.full((pad,), TRASH, jnp.int32)]).reshape(EROWS, LANES)

    r2 = lambda b: b.reshape(1, -1)

    m1 = _tc_call(_t1_body, [F_IN], [(F_IN, H), (1, H), (H, H), (1, H)], [H])(
        x, W1a, r2(b1a), W1b, r2(b1b))
    s1p, (c0, c1) = _edge_scatter(m1, rows_q, cols_p, first=True)

    u, m2 = _tc_call(
        _t2_body, [F_IN, HQ, HQ, 1, 1],
        [(F_IN, H), (H, H), (1, H), (H, H), (1, H),
         (F_IN, H), (H, H), (1, H), (H, H), (1, H)],
        [H, H],
    )(x, *s1p, c0, c1,
      W1c[:F_IN], W1c[F_IN:], r2(b1c), W1d, r2(b1d),
      W2a[:F_IN], W2a[F_IN:], r2(b2a), W2b, r2(b2b))

    s2p, _ = _edge_scatter(m2, rows_q, cols_p, first=False)

    out = _tc_call(
        _t3_body, [F_IN, H, HQ, HQ, 1, 1],
        [(F_IN, H), (H, H), (H, H), (1, H), (H, OUT), (1, OUT)],
        [OUT],
    )(x, u, *s2p, c0, c1,
      W2c[:F_IN], W2c[F_IN:F_IN + H], W2c[F_IN + H:], r2(b2c), W2d, r2(b2d))
    return out


# speedup vs baseline: 8.3543x; 1.1041x over previous
import functools

import jax
import jax.numpy as jnp
from jax import lax
from jax.experimental import pallas as pl
from jax.experimental.pallas import tpu as pltpu
from jax.experimental.pallas import tpu_sc as plsc

N = 50000
E = 800000
F_IN = 128
H = 64
OUT = 2
HQ = 32
NQ = H // HQ

NC = 2
NS = 16
LANES = 128
NCH = 392
G = 14
NG = NCH // G
EP = NS * NCH * LANES
EROWS = EP // LANES
TRASH = N
ACC_ROWS = 51200
STRIPE = ACC_ROWS // NS
ZROWS = 64

BM = 1024

_f32 = jnp.float32


@functools.lru_cache(maxsize=None)
def _make_sc_pass(with_cnt: bool):
    mesh = plsc.VectorSubcoreMesh(core_axis_name="c", subcore_axis_name="s",
                                  num_cores=NC, num_subcores=NS)
    out_type = [jax.ShapeDtypeStruct((NC, ACC_ROWS, HQ), _f32)]
    scratch = [
        pltpu.VMEM((2, G, LANES), jnp.int32),
        pltpu.VMEM((2, G, LANES), jnp.int32),
        pltpu.VMEM((2, LANES, HQ), _f32),
        pltpu.VMEM((ZROWS, HQ), _f32),
        pltpu.MemorySpace.VMEM_SHARED((ACC_ROWS, HQ), _f32),
        pltpu.SemaphoreType.DMA,
        pltpu.SemaphoreType.DMA,
        pltpu.SemaphoreType.DMA,
    ]
    if with_cnt:
        out_type.append(jax.ShapeDtypeStruct((NC, ACC_ROWS), _f32))
        scratch += [
            pltpu.VMEM((LANES,), _f32),
            pltpu.VMEM((LANES,), _f32),
            pltpu.MemorySpace.VMEM_SHARED((ACC_ROWS,), _f32),
        ]

    def body(m_lo, m_hi, rows, cols, *refs):
        if with_cnt:
            (s_out, cnt_out, idxr_v, idxc_v, val_v, zbuf, acc_sh, sem0, sem1,
             isem, ones_v, zrow, cnt_sh) = refs
        else:
            (s_out, idxr_v, idxc_v, val_v, zbuf, acc_sh, sem0, sem1, isem) = refs
        c = lax.axis_index("c")
        s = lax.axis_index("s")
        base = s * NCH
        stripe0 = s * STRIPE

        def _stage(g, slot):
            pltpu.async_copy(rows.at[pl.ds(base + g * G, G)],
                             idxr_v.at[slot], isem)
            pltpu.async_copy(cols.at[pl.ds(base + g * G, G)],
                             idxc_v.at[slot], isem)

        def _stage_wait(slot):
            pltpu.make_async_copy(rows.at[pl.ds(base, G)], idxr_v.at[slot],
                                  isem).wait()
            pltpu.make_async_copy(cols.at[pl.ds(base, G)], idxc_v.at[slot],
                                  isem).wait()

        def _zfill(i, carry):
            zbuf[i, pl.ds(0, 16)] = jnp.zeros((16,), _f32)
            zbuf[i, pl.ds(16, 16)] = jnp.zeros((16,), _f32)
            return carry

        lax.fori_loop(0, ZROWS, _zfill, 0)
        if with_cnt:
            for k in range(LANES // 16):
                ones_v[pl.ds(k * 16, 16)] = jnp.ones((16,), _f32)
                zrow[pl.ds(k * 16, 16)] = jnp.zeros((16,), _f32)

        _stage(0, 0)

        def _zacc(j, carry):
            pltpu.sync_copy(zbuf, acc_sh.at[pl.ds(stripe0 + j * ZROWS, ZROWS)])
            return carry

        lax.fori_loop(0, STRIPE // ZROWS, _zacc, 0)
        if with_cnt:
            def _zcnt(j, carry):
                pltpu.sync_copy(zrow, cnt_sh.at[pl.ds(stripe0 + j * LANES, LANES)])
                return carry

            lax.fori_loop(0, STRIPE // LANES, _zcnt, 0)
        plsc.subcore_barrier()

        def _start(slot, j, buf, sem):
            @pl.when(c == 0)
            def _():
                pltpu.async_copy(m_lo.at[idxr_v.at[slot, j]], val_v.at[buf], sem)

            @pl.when(c == 1)
            def _():
                pltpu.async_copy(m_hi.at[idxr_v.at[slot, j]], val_v.at[buf], sem)

        def _wait(buf, sem):
            pltpu.make_async_copy(m_lo.at[idxr_v.at[0, 0]], val_v.at[buf],
                                  sem).wait()

        def _scatter(slot, j, buf):
            pltpu.sync_copy(val_v.at[buf], acc_sh.at[idxc_v.at[slot, j]],
                            add=True)

        def _count(g, slot, j):
            do = ((c == 0) & (g < NG // 2)) | ((c == 1) & (g >= NG // 2))

            @pl.when(do)
            def _():
                pltpu.sync_copy(ones_v, cnt_sh.at[idxc_v.at[slot, j]], add=True)

        def _group(g, carry):
            slot = jnp.bitwise_and(g, 1)
            _stage_wait(slot)

            @pl.when(g + 1 < NG)
            def _():
                _stage(g + 1, 1 - slot)

            _start(slot, 0, 0, sem0)
            for j in range(0, G, 2):
                if j + 1 < G:
                    _start(slot, j + 1, 1, sem1)
                _wait(0, sem0)
                _scatter(slot, j, 0)
                if with_cnt:
                    _count(g, slot, j)
                if j + 2 < G:
                    _start(slot, j + 2, 0, sem0)
                if j + 1 < G:
                    _wait(1, sem1)
                    _scatter(slot, j + 1, 1)
                    if with_cnt:
                        _count(g, slot, j + 1)
            return carry

        lax.fori_loop(0, NG, _group, 0)
        plsc.subcore_barrier()

        pltpu.sync_copy(acc_sh.at[pl.ds(stripe0, STRIPE)],
                        s_out.at[c, pl.ds(stripe0, STRIPE)])
        if with_cnt:
            pltpu.sync_copy(cnt_sh.at[pl.ds(stripe0, STRIPE)],
                            cnt_out.at[c, pl.ds(stripe0, STRIPE)])

    return pl.kernel(
        body, out_type=out_type, mesh=mesh, scratch_types=scratch,
        compiler_params=pltpu.CompilerParams(use_tc_tiling_on_sc=False))


def _dot(a, b):
    return jnp.dot(a, b, preferred_element_type=_f32)


def _t1_body(x_ref, wa, ba, wb, bb, lo_ref, hi_ref):
    h = jnp.maximum(_dot(x_ref[...], wa[...]) + ba[...], 0.0)
    m = _dot(h, wb[...]) + bb[...]
    lo_ref[...] = m[:, :HQ]
    hi_ref[...] = m[:, HQ:]


def _t2_body(x_ref, s0, s1, c0, c1, w1c0, w1c1, b1c, w1d, b1d,
             w2a0, w2a1, b2a, w2b, b2b, u_ref, lo_ref, hi_ref):
    cnt = jnp.maximum(c0[...] + c1[...], 1.0)
    agg = jnp.concatenate([s0[...], s1[...]], axis=1) / cnt
    t = _dot(x_ref[...], w1c0[...]) + _dot(agg, w1c1[...]) + b1c[...]
    u = _dot(jnp.maximum(t, 0.0), w1d[...]) + b1d[...]
    u_ref[...] = u
    t2 = _dot(x_ref[...], w2a0[...]) + _dot(u, w2a1[...]) + b2a[...]
    m2 = _dot(jnp.maximum(t2, 0.0), w2b[...]) + b2b[...]
    lo_ref[...] = m2[:, :HQ]
    hi_ref[...] = m2[:, HQ:]


def _t3_body(x_ref, u, s0, s1, c0, c1, w2c0, w2c1, w2c2, b2c, wd, bd,
             o_ref):
    cnt = jnp.maximum(c0[...] + c1[...], 1.0)
    agg = jnp.concatenate([s0[...], s1[...]], axis=1) / cnt
    t = (_dot(x_ref[...], w2c0[...]) + _dot(u[...], w2c1[...])
         + _dot(agg, w2c2[...]) + b2c[...])
    o_ref[...] = _dot(jnp.maximum(t, 0.0), wd[...]) + bd[...]


def _row_spec(width):
    return pl.BlockSpec((BM, width), lambda i: (i, 0))


def _full_spec(shape):
    return pl.BlockSpec(shape, lambda i: (0,) * len(shape))


def _tc_call(body, in_widths, full_shapes, out_widths):
    grid = (pl.cdiv(N, BM),)
    out_shape = [jax.ShapeDtypeStruct((N, w), _f32) for w in out_widths]
    out_specs = [_row_spec(w) for w in out_widths]
    if len(out_widths) == 1:
        out_shape, out_specs = out_shape[0], out_specs[0]
    return pl.pallas_call(
        body,
        grid=grid,
        in_specs=[_row_spec(w) for w in in_widths] + [_full_spec(s) for s in full_shapes],
        out_specs=out_specs,
        out_shape=out_shape,
    )


def _edge_scatter(m_lo, m_hi, rows_p, cols_p, first):
    if first:
        sa, cnt = _make_sc_pass(True)(m_lo, m_hi, rows_p, cols_p)
        return ((sa[0, :N], sa[1, :N]),
                (cnt[0, :N].reshape(N, 1), cnt[1, :N].reshape(N, 1)))
    (sa,) = _make_sc_pass(False)(m_lo, m_hi, rows_p, cols_p)
    return (sa[0, :N], sa[1, :N]), None


def kernel(x, edge_index, edge_attr, W1a, b1a, W1b, b1b, W1c, b1c, W1d, b1d,
           W2a, b2a, W2b, b2b, W2c, b2c, W2d, b2d):
    del edge_attr
    row = edge_index[0].astype(jnp.int32)
    col = edge_index[1].astype(jnp.int32)
    pad = EP - E
    rows_p = jnp.concatenate([row, jnp.zeros((pad,), jnp.int32)]).reshape(EROWS, LANES)
    cols_p = jnp.concatenate([col, jnp.full((pad,), TRASH, jnp.int32)]).reshape(EROWS, LANES)

    r2 = lambda b: b.reshape(1, -1)

    m1_lo, m1_hi = _tc_call(
        _t1_body, [F_IN], [(F_IN, H), (1, H), (H, H), (1, H)], [HQ, HQ])(
        x, W1a, r2(b1a), W1b, r2(b1b))
    s1p, (c0, c1) = _edge_scatter(m1_lo, m1_hi, rows_p, cols_p, first=True)

    u, m2_lo, m2_hi = _tc_call(
        _t2_body, [F_IN, HQ, HQ, 1, 1],
        [(F_IN, H), (H, H), (1, H), (H, H), (1, H),
         (F_IN, H), (H, H), (1, H), (H, H), (1, H)],
        [H, HQ, HQ],
    )(x, *s1p, c0, c1,
      W1c[:F_IN], W1c[F_IN:], r2(b1c), W1d, r2(b1d),
      W2a[:F_IN], W2a[F_IN:], r2(b2a), W2b, r2(b2b))

    s2p, _ = _edge_scatter(m2_lo, m2_hi, rows_p, cols_p, first=False)

    out = _tc_call(
        _t3_body, [F_IN, H, HQ, HQ, 1, 1],
        [(F_IN, H), (H, H), (H, H), (1, H), (H, OUT), (1, OUT)],
        [OUT],
    )(x, u, *s2p, c0, c1,
      W2c[:F_IN], W2c[F_IN:F_IN + H], W2c[F_IN + H:], r2(b2c), W2d, r2(b2d))
    return out


# speedup vs baseline: 9.2211x; 1.1038x over previous
import functools

import jax
import jax.numpy as jnp
from jax import lax
from jax.experimental import pallas as pl
from jax.experimental.pallas import tpu as pltpu
from jax.experimental.pallas import tpu_sc as plsc

N = 50000
E = 800000
F_IN = 128
H = 64
OUT = 2
HQ = 32
NQ = H // HQ

NC = 2
NS = 16
LANES = 128
NCH = 392
G = 8
NG = NCH // G
R = 4
EP = NS * NCH * LANES
EROWS = EP // LANES
TRASH = N
ACC_ROWS = 50176
STRIPE = ACC_ROWS // NS
ZROWS = 32

BM = 1024

_f32 = jnp.float32


@functools.lru_cache(maxsize=None)
def _make_sc_pass(with_cnt: bool):
    mesh = plsc.VectorSubcoreMesh(core_axis_name="c", subcore_axis_name="s",
                                  num_cores=NC, num_subcores=NS)
    out_type = [jax.ShapeDtypeStruct((NC, ACC_ROWS, HQ), _f32)]
    scratch = [
        pltpu.VMEM((2, G, LANES), jnp.int32),
        pltpu.VMEM((2, G, LANES), jnp.int32),
        pltpu.VMEM((R, LANES, HQ), _f32),
        pltpu.VMEM((ZROWS, HQ), _f32),
        pltpu.MemorySpace.VMEM_SHARED((ACC_ROWS, HQ), _f32),
        [pltpu.SemaphoreType.DMA] * R,
        [pltpu.SemaphoreType.DMA] * R,
        pltpu.SemaphoreType.DMA,
    ]
    if with_cnt:
        out_type.append(jax.ShapeDtypeStruct((NC, ACC_ROWS), _f32))
        scratch += [
            pltpu.VMEM((LANES,), _f32),
            pltpu.VMEM((ZROWS,), _f32),
            pltpu.MemorySpace.VMEM_SHARED((ACC_ROWS,), _f32),
        ]

    def body(m_lo, m_hi, rows, cols, *refs):
        if with_cnt:
            (s_out, cnt_out, idxr_v, idxc_v, val_v, zbuf, acc_sh, gsem, ssem,
             isem, ones_v, zrow, cnt_sh) = refs
        else:
            (s_out, idxr_v, idxc_v, val_v, zbuf, acc_sh, gsem, ssem,
             isem) = refs
        c = lax.axis_index("c")
        s = lax.axis_index("s")
        base = s * NCH
        stripe0 = s * STRIPE

        def _stage(g, slot):
            pltpu.async_copy(rows.at[pl.ds(base + g * G, G)],
                             idxr_v.at[slot], isem)
            pltpu.async_copy(cols.at[pl.ds(base + g * G, G)],
                             idxc_v.at[slot], isem)

        def _stage_wait(slot):
            pltpu.make_async_copy(rows.at[pl.ds(base, G)], idxr_v.at[slot],
                                  isem).wait()
            pltpu.make_async_copy(cols.at[pl.ds(base, G)], idxc_v.at[slot],
                                  isem).wait()

        def _zfill(i, carry):
            zbuf[i, pl.ds(0, 16)] = jnp.zeros((16,), _f32)
            zbuf[i, pl.ds(16, 16)] = jnp.zeros((16,), _f32)
            return carry

        lax.fori_loop(0, ZROWS, _zfill, 0)
        if with_cnt:
            for k in range(LANES // 16):
                ones_v[pl.ds(k * 16, 16)] = jnp.ones((16,), _f32)
            for k in range(ZROWS // 16):
                zrow[pl.ds(k * 16, 16)] = jnp.zeros((16,), _f32)

        _stage(0, 0)

        def _zacc(j, carry):
            pltpu.sync_copy(zbuf, acc_sh.at[pl.ds(stripe0 + j * ZROWS, ZROWS)])
            return carry

        lax.fori_loop(0, STRIPE // ZROWS, _zacc, 0)
        if with_cnt:
            def _zcnt(j, carry):
                pltpu.sync_copy(zrow, cnt_sh.at[pl.ds(stripe0 + j * ZROWS, ZROWS)])
                return carry

            lax.fori_loop(0, STRIPE // ZROWS, _zcnt, 0)
        plsc.subcore_barrier()

        def _gather(slot, j, buf):
            @pl.when(c == 0)
            def _():
                pltpu.async_copy(m_lo.at[idxr_v.at[slot, j]], val_v.at[buf],
                                 gsem[buf])

            @pl.when(c == 1)
            def _():
                pltpu.async_copy(m_hi.at[idxr_v.at[slot, j]], val_v.at[buf],
                                 gsem[buf])

        def _gwait(buf):
            pltpu.make_async_copy(m_lo.at[idxr_v.at[0, 0]], val_v.at[buf],
                                  gsem[buf]).wait()

        def _scatter(slot, j, buf):
            pltpu.async_copy(val_v.at[buf], acc_sh.at[idxc_v.at[slot, j]],
                             ssem[buf], add=True)

        def _swait(buf):
            pltpu.make_async_copy(val_v.at[buf], acc_sh.at[idxc_v.at[0, 0]],
                                  ssem[buf]).wait()

        def _count(g, slot, j):
            do = ((c == 0) & (g < NG // 2)) | ((c == 1) & (g >= NG // 2))

            @pl.when(do)
            def _():
                pltpu.sync_copy(ones_v, cnt_sh.at[idxc_v.at[slot, j]], add=True)

        def _group(g, carry):
            slot = jnp.bitwise_and(g, 1)
            _stage_wait(slot)

            @pl.when(g + 1 < NG)
            def _():
                _stage(g + 1, 1 - slot)

            for j in range(G):
                buf = j % R
                if j < R:
                    @pl.when(g > 0)
                    def _():
                        _swait(buf)
                else:
                    _swait(buf)
                _gather(slot, j, buf)
                if j >= 2:
                    _gwait((j - 2) % R)
                    _scatter(slot, j - 2, (j - 2) % R)
                    if with_cnt:
                        _count(g, slot, j - 2)
            for j in range(G - 2, G):
                _gwait(j % R)
                _scatter(slot, j, j % R)
                if with_cnt:
                    _count(g, slot, j)
            return carry

        lax.fori_loop(0, NG, _group, 0)
        for buf in range(R):
            _swait(buf)
        plsc.subcore_barrier()

        pltpu.sync_copy(acc_sh.at[pl.ds(stripe0, STRIPE)],
                        s_out.at[c, pl.ds(stripe0, STRIPE)])
        if with_cnt:
            pltpu.sync_copy(cnt_sh.at[pl.ds(stripe0, STRIPE)],
                            cnt_out.at[c, pl.ds(stripe0, STRIPE)])

    return pl.kernel(
        body, out_type=out_type, mesh=mesh, scratch_types=scratch,
        compiler_params=pltpu.CompilerParams(use_tc_tiling_on_sc=False))


def _dot(a, b):
    return jnp.dot(a, b, preferred_element_type=_f32)


def _t1_body(x_ref, wa, ba, wb, bb, lo_ref, hi_ref):
    h = jnp.maximum(_dot(x_ref[...], wa[...]) + ba[...], 0.0)
    m = _dot(h, wb[...]) + bb[...]
    lo_ref[...] = m[:, :HQ]
    hi_ref[...] = m[:, HQ:]


def _t2_body(x_ref, s0, s1, c0, c1, w1c0, w1c1, b1c, w1d, b1d,
             w2a0, w2a1, b2a, w2b, b2b, u_ref, lo_ref, hi_ref):
    cnt = jnp.maximum(c0[...] + c1[...], 1.0)
    agg = jnp.concatenate([s0[...], s1[...]], axis=1) / cnt
    t = _dot(x_ref[...], w1c0[...]) + _dot(agg, w1c1[...]) + b1c[...]
    u = _dot(jnp.maximum(t, 0.0), w1d[...]) + b1d[...]
    u_ref[...] = u
    t2 = _dot(x_ref[...], w2a0[...]) + _dot(u, w2a1[...]) + b2a[...]
    m2 = _dot(jnp.maximum(t2, 0.0), w2b[...]) + b2b[...]
    lo_ref[...] = m2[:, :HQ]
    hi_ref[...] = m2[:, HQ:]


def _t3_body(x_ref, u, s0, s1, c0, c1, w2c0, w2c1, w2c2, b2c, wd, bd,
             o_ref):
    cnt = jnp.maximum(c0[...] + c1[...], 1.0)
    agg = jnp.concatenate([s0[...], s1[...]], axis=1) / cnt
    t = (_dot(x_ref[...], w2c0[...]) + _dot(u[...], w2c1[...])
         + _dot(agg, w2c2[...]) + b2c[...])
    o_ref[...] = _dot(jnp.maximum(t, 0.0), wd[...]) + bd[...]


def _row_spec(width):
    return pl.BlockSpec((BM, width), lambda i: (i, 0))


def _full_spec(shape):
    return pl.BlockSpec(shape, lambda i: (0,) * len(shape))


def _tc_call(body, in_widths, full_shapes, out_widths):
    grid = (pl.cdiv(N, BM),)
    out_shape = [jax.ShapeDtypeStruct((N, w), _f32) for w in out_widths]
    out_specs = [_row_spec(w) for w in out_widths]
    if len(out_widths) == 1:
        out_shape, out_specs = out_shape[0], out_specs[0]
    return pl.pallas_call(
        body,
        grid=grid,
        in_specs=[_row_spec(w) for w in in_widths] + [_full_spec(s) for s in full_shapes],
        out_specs=out_specs,
        out_shape=out_shape,
    )


def _edge_scatter(m_lo, m_hi, rows_p, cols_p, first):
    if first:
        sa, cnt = _make_sc_pass(True)(m_lo, m_hi, rows_p, cols_p)
        return ((sa[0, :N], sa[1, :N]),
                (cnt[0, :N].reshape(N, 1), cnt[1, :N].reshape(N, 1)))
    (sa,) = _make_sc_pass(False)(m_lo, m_hi, rows_p, cols_p)
    return (sa[0, :N], sa[1, :N]), None


def kernel(x, edge_index, edge_attr, W1a, b1a, W1b, b1b, W1c, b1c, W1d, b1d,
           W2a, b2a, W2b, b2b, W2c, b2c, W2d, b2d):
    del edge_attr
    row = edge_index[0].astype(jnp.int32)
    col = edge_index[1].astype(jnp.int32)
    pad = EP - E
    rows_p = jnp.concatenate([row, jnp.zeros((pad,), jnp.int32)]).reshape(EROWS, LANES)
    cols_p = jnp.concatenate([col, jnp.full((pad,), TRASH, jnp.int32)]).reshape(EROWS, LANES)

    r2 = lambda b: b.reshape(1, -1)

    m1_lo, m1_hi = _tc_call(
        _t1_body, [F_IN], [(F_IN, H), (1, H), (H, H), (1, H)], [HQ, HQ])(
        x, W1a, r2(b1a), W1b, r2(b1b))
    s1p, (c0, c1) = _edge_scatter(m1_lo, m1_hi, rows_p, cols_p, first=True)

    u, m2_lo, m2_hi = _tc_call(
        _t2_body, [F_IN, HQ, HQ, 1, 1],
        [(F_IN, H), (H, H), (1, H), (H, H), (1, H),
         (F_IN, H), (H, H), (1, H), (H, H), (1, H)],
        [H, HQ, HQ],
    )(x, *s1p, c0, c1,
      W1c[:F_IN], W1c[F_IN:], r2(b1c), W1d, r2(b1d),
      W2a[:F_IN], W2a[F_IN:], r2(b2a), W2b, r2(b2b))

    s2p, _ = _edge_scatter(m2_lo, m2_hi, rows_p, cols_p, first=False)

    out = _tc_call(
        _t3_body, [F_IN, H, HQ, HQ, 1, 1],
        [(F_IN, H), (H, H), (H, H), (1, H), (H, OUT), (1, OUT)],
        [OUT],
    )(x, u, *s2p, c0, c1,
      W2c[:F_IN], W2c[F_IN:F_IN + H], W2c[F_IN + H:], r2(b2c), W2d, r2(b2d))
    return out


# speedup vs baseline: 10.2011x; 1.1063x over previous
import functools

import jax
import jax.numpy as jnp
from jax import lax
from jax.experimental import pallas as pl
from jax.experimental.pallas import tpu as pltpu
from jax.experimental.pallas import tpu_sc as plsc

N = 50000
E = 800000
F_IN = 128
H = 64
OUT = 2
HQ = 32
NQ = H // HQ

NC = 2
NS = 16
LANES = 128
NCH = 392
G = 8
NG = NCH // G
R = 4
EP = NS * NCH * LANES
EROWS = EP // LANES
TRASH = N
ACC_ROWS = 50176
STRIPE = ACC_ROWS // NS
ZROWS = 32

BM = 2048

_f32 = jnp.float32


@functools.lru_cache(maxsize=None)
def _make_sc_pass(with_cnt: bool):
    mesh = plsc.VectorSubcoreMesh(core_axis_name="c", subcore_axis_name="s",
                                  num_cores=NC, num_subcores=NS)
    out_type = [jax.ShapeDtypeStruct((NC, ACC_ROWS, HQ), _f32)]
    scratch = [
        pltpu.VMEM((2, G, LANES), jnp.int32),
        pltpu.VMEM((2, G, LANES), jnp.int32),
        pltpu.VMEM((R, LANES, HQ), _f32),
        pltpu.VMEM((ZROWS, HQ), _f32),
        pltpu.MemorySpace.VMEM_SHARED((ACC_ROWS, HQ), _f32),
        [pltpu.SemaphoreType.DMA] * R,
        [pltpu.SemaphoreType.DMA] * R,
        pltpu.SemaphoreType.DMA,
    ]
    if with_cnt:
        out_type.append(jax.ShapeDtypeStruct((NC, ACC_ROWS), _f32))
        scratch += [
            pltpu.VMEM((LANES,), _f32),
            pltpu.VMEM((ZROWS,), _f32),
            pltpu.MemorySpace.VMEM_SHARED((ACC_ROWS,), _f32),
        ]

    def body(m_lo, m_hi, rows, cols, *refs):
        if with_cnt:
            (s_out, cnt_out, idxr_v, idxc_v, val_v, zbuf, acc_sh, gsem, ssem,
             isem, ones_v, zrow, cnt_sh) = refs
        else:
            (s_out, idxr_v, idxc_v, val_v, zbuf, acc_sh, gsem, ssem,
             isem) = refs
        c = lax.axis_index("c")
        s = lax.axis_index("s")
        base = s * NCH
        stripe0 = s * STRIPE

        def _stage(g, slot):
            pltpu.async_copy(rows.at[pl.ds(base + g * G, G)],
                             idxr_v.at[slot], isem)
            pltpu.async_copy(cols.at[pl.ds(base + g * G, G)],
                             idxc_v.at[slot], isem)

        def _stage_wait(slot):
            pltpu.make_async_copy(rows.at[pl.ds(base, G)], idxr_v.at[slot],
                                  isem).wait()
            pltpu.make_async_copy(cols.at[pl.ds(base, G)], idxc_v.at[slot],
                                  isem).wait()

        def _zfill(i, carry):
            zbuf[i, pl.ds(0, 16)] = jnp.zeros((16,), _f32)
            zbuf[i, pl.ds(16, 16)] = jnp.zeros((16,), _f32)
            return carry

        lax.fori_loop(0, ZROWS, _zfill, 0)
        if with_cnt:
            for k in range(LANES // 16):
                ones_v[pl.ds(k * 16, 16)] = jnp.ones((16,), _f32)
            for k in range(ZROWS // 16):
                zrow[pl.ds(k * 16, 16)] = jnp.zeros((16,), _f32)

        _stage(0, 0)

        def _zacc(j, carry):
            pltpu.sync_copy(zbuf, acc_sh.at[pl.ds(stripe0 + j * ZROWS, ZROWS)])
            return carry

        lax.fori_loop(0, STRIPE // ZROWS, _zacc, 0)
        if with_cnt:
            def _zcnt(j, carry):
                pltpu.sync_copy(zrow, cnt_sh.at[pl.ds(stripe0 + j * ZROWS, ZROWS)])
                return carry

            lax.fori_loop(0, STRIPE // ZROWS, _zcnt, 0)
        plsc.subcore_barrier()

        def _gather(slot, j, buf):
            @pl.when(c == 0)
            def _():
                pltpu.async_copy(m_lo.at[idxr_v.at[slot, j]], val_v.at[buf],
                                 gsem[buf])

            @pl.when(c == 1)
            def _():
                pltpu.async_copy(m_hi.at[idxr_v.at[slot, j]], val_v.at[buf],
                                 gsem[buf])

        def _gwait(buf):
            pltpu.make_async_copy(m_lo.at[idxr_v.at[0, 0]], val_v.at[buf],
                                  gsem[buf]).wait()

        def _scatter(slot, j, buf):
            pltpu.async_copy(val_v.at[buf], acc_sh.at[idxc_v.at[slot, j]],
                             ssem[buf], add=True)

        def _swait(buf):
            pltpu.make_async_copy(val_v.at[buf], acc_sh.at[idxc_v.at[0, 0]],
                                  ssem[buf]).wait()

        def _count(g, slot, j):
            do = ((c == 0) & (g < NG // 2)) | ((c == 1) & (g >= NG // 2))

            @pl.when(do)
            def _():
                pltpu.sync_copy(ones_v, cnt_sh.at[idxc_v.at[slot, j]], add=True)

        def _group(g, carry):
            slot = jnp.bitwise_and(g, 1)
            _stage_wait(slot)

            @pl.when(g + 1 < NG)
            def _():
                _stage(g + 1, 1 - slot)

            for j in range(G):
                buf = j % R
                if j < R:
                    @pl.when(g > 0)
                    def _():
                        _swait(buf)
                else:
                    _swait(buf)
                _gather(slot, j, buf)
                if j >= 2:
                    _gwait((j - 2) % R)
                    _scatter(slot, j - 2, (j - 2) % R)
                    if with_cnt:
                        _count(g, slot, j - 2)
            for j in range(G - 2, G):
                _gwait(j % R)
                _scatter(slot, j, j % R)
                if with_cnt:
                    _count(g, slot, j)
            return carry

        lax.fori_loop(0, NG, _group, 0)
        for buf in range(R):
            _swait(buf)
        plsc.subcore_barrier()

        pltpu.sync_copy(acc_sh.at[pl.ds(stripe0, STRIPE)],
                        s_out.at[c, pl.ds(stripe0, STRIPE)])
        if with_cnt:
            pltpu.sync_copy(cnt_sh.at[pl.ds(stripe0, STRIPE)],
                            cnt_out.at[c, pl.ds(stripe0, STRIPE)])

    return pl.kernel(
        body, out_type=out_type, mesh=mesh, scratch_types=scratch,
        compiler_params=pltpu.CompilerParams(use_tc_tiling_on_sc=False))


def _dot(a, b):
    return jnp.dot(a, b, preferred_element_type=_f32)


def _t1_body(x_ref, wa, ba, wb, bb, lo_ref, hi_ref):
    h = jnp.maximum(_dot(x_ref[...], wa[...]) + ba[...], 0.0)
    m = _dot(h, wb[...]) + bb[...]
    lo_ref[...] = m[:, :HQ]
    hi_ref[...] = m[:, HQ:]


def _t2_body(x_ref, s0, s1, c0, c1, w1c0, w1c1, b1c, w1d, b1d,
             w2a0, w2a1, b2a, w2b, b2b, u_ref, lo_ref, hi_ref):
    cnt = jnp.maximum(c0[...] + c1[...], 1.0).reshape(-1, 1)
    agg = jnp.concatenate([s0[...], s1[...]], axis=1) / cnt
    t = _dot(x_ref[...], w1c0[...]) + _dot(agg, w1c1[...]) + b1c[...]
    u = _dot(jnp.maximum(t, 0.0), w1d[...]) + b1d[...]
    u_ref[...] = u
    t2 = _dot(x_ref[...], w2a0[...]) + _dot(u, w2a1[...]) + b2a[...]
    m2 = _dot(jnp.maximum(t2, 0.0), w2b[...]) + b2b[...]
    lo_ref[...] = m2[:, :HQ]
    hi_ref[...] = m2[:, HQ:]


def _t3_body(x_ref, u, s0, s1, c0, c1, w2c0, w2c1, w2c2, b2c, wd, bd,
             o_ref):
    cnt = jnp.maximum(c0[...] + c1[...], 1.0).reshape(-1, 1)
    agg = jnp.concatenate([s0[...], s1[...]], axis=1) / cnt
    t = (_dot(x_ref[...], w2c0[...]) + _dot(u[...], w2c1[...])
         + _dot(agg, w2c2[...]) + b2c[...])
    o_ref[...] = _dot(jnp.maximum(t, 0.0), wd[...]) + bd[...]


def _row_spec(width):
    if width is None:
        return pl.BlockSpec((BM,), lambda i: (i,))
    return pl.BlockSpec((BM, width), lambda i: (i, 0))


def _full_spec(shape):
    return pl.BlockSpec(shape, lambda i: (0,) * len(shape))


def _tc_call(body, in_widths, full_shapes, out_widths):
    grid = (pl.cdiv(N, BM),)
    out_shape = [jax.ShapeDtypeStruct((N, w), _f32) for w in out_widths]
    out_specs = [_row_spec(w) for w in out_widths]
    if len(out_widths) == 1:
        out_shape, out_specs = out_shape[0], out_specs[0]
    return pl.pallas_call(
        body,
        grid=grid,
        in_specs=[_row_spec(w) for w in in_widths] + [_full_spec(s) for s in full_shapes],
        out_specs=out_specs,
        out_shape=out_shape,
    )


def _edge_scatter(m_lo, m_hi, rows_p, cols_p, first):
    if first:
        sa, cnt = _make_sc_pass(True)(m_lo, m_hi, rows_p, cols_p)
        return (sa[0, :N], sa[1, :N]), (cnt[0], cnt[1])
    (sa,) = _make_sc_pass(False)(m_lo, m_hi, rows_p, cols_p)
    return (sa[0, :N], sa[1, :N]), None


def kernel(x, edge_index, edge_attr, W1a, b1a, W1b, b1b, W1c, b1c, W1d, b1d,
           W2a, b2a, W2b, b2b, W2c, b2c, W2d, b2d):
    del edge_attr
    row = edge_index[0].astype(jnp.int32)
    col = edge_index[1].astype(jnp.int32)
    pad = EP - E
    rows_p = jnp.concatenate([row, jnp.zeros((pad,), jnp.int32)]).reshape(EROWS, LANES)
    cols_p = jnp.concatenate([col, jnp.full((pad,), TRASH, jnp.int32)]).reshape(EROWS, LANES)

    r2 = lambda b: b.reshape(1, -1)

    m1_lo, m1_hi = _tc_call(
        _t1_body, [F_IN], [(F_IN, H), (1, H), (H, H), (1, H)], [HQ, HQ])(
        x, W1a, r2(b1a), W1b, r2(b1b))
    s1p, (c0, c1) = _edge_scatter(m1_lo, m1_hi, rows_p, cols_p, first=True)

    u, m2_lo, m2_hi = _tc_call(
        _t2_body, [F_IN, HQ, HQ, None, None],
        [(F_IN, H), (H, H), (1, H), (H, H), (1, H),
         (F_IN, H), (H, H), (1, H), (H, H), (1, H)],
        [H, HQ, HQ],
    )(x, *s1p, c0, c1,
      W1c[:F_IN], W1c[F_IN:], r2(b1c), W1d, r2(b1d),
      W2a[:F_IN], W2a[F_IN:], r2(b2a), W2b, r2(b2b))

    s2p, _ = _edge_scatter(m2_lo, m2_hi, rows_p, cols_p, first=False)

    out = _tc_call(
        _t3_body, [F_IN, H, HQ, HQ, None, None],
        [(F_IN, H), (H, H), (H, H), (1, H), (H, OUT), (1, OUT)],
        [OUT],
    )(x, u, *s2p, c0, c1,
      W2c[:F_IN], W2c[F_IN:F_IN + H], W2c[F_IN + H:], r2(b2c), W2d, r2(b2d))
    return out


# speedup vs baseline: 10.4618x; 1.0255x over previous
import functools

import jax
import jax.numpy as jnp
from jax import lax
from jax.experimental import pallas as pl
from jax.experimental.pallas import tpu as pltpu
from jax.experimental.pallas import tpu_sc as plsc

N = 50000
E = 800000
F_IN = 128
H = 64
OUT = 2
HQ = 32
NQ = H // HQ

NC = 2
NS = 16
LANES = 128
NCH = 392
G = 8
NG = NCH // G
R = 4
EP = NS * NCH * LANES
EROWS = EP // LANES
TRASH = N
ACC_ROWS = 50176
STRIPE = ACC_ROWS // NS
ZROWS = 32

BM = 2048

_f32 = jnp.float32


@functools.lru_cache(maxsize=None)
def _make_sc_pass(with_cnt: bool):
    mesh = plsc.VectorSubcoreMesh(core_axis_name="c", subcore_axis_name="s",
                                  num_cores=NC, num_subcores=NS)
    out_type = [jax.ShapeDtypeStruct((NC, ACC_ROWS, HQ), _f32)]
    scratch = [
        pltpu.VMEM((2, G, LANES), jnp.int32),
        pltpu.VMEM((2, G, LANES), jnp.int32),
        pltpu.VMEM((R, LANES, HQ), _f32),
        pltpu.VMEM((ZROWS, HQ), _f32),
        pltpu.MemorySpace.VMEM_SHARED((ACC_ROWS, HQ), _f32),
        [pltpu.SemaphoreType.DMA] * R,
        [pltpu.SemaphoreType.DMA] * R,
        pltpu.SemaphoreType.DMA,
        pltpu.SemaphoreType.DMA,
    ]
    if with_cnt:
        out_type.append(jax.ShapeDtypeStruct((NC, ACC_ROWS), _f32))
        scratch += [
            pltpu.VMEM((LANES,), _f32),
            pltpu.VMEM((ZROWS,), _f32),
            pltpu.MemorySpace.VMEM_SHARED((ACC_ROWS,), _f32),
            pltpu.SemaphoreType.DMA,
        ]

    def body(m_lo, m_hi, rows, cols, *refs):
        if with_cnt:
            (s_out, cnt_out, idxr_v, idxc_v, val_v, zbuf, acc_sh, gsem, ssem,
             isem, zsem, ones_v, zrow, cnt_sh, csem) = refs
        else:
            (s_out, idxr_v, idxc_v, val_v, zbuf, acc_sh, gsem, ssem,
             isem, zsem) = refs
        c = lax.axis_index("c")
        s = lax.axis_index("s")
        base = s * NCH
        stripe0 = s * STRIPE

        def _stage(g, slot):
            pltpu.async_copy(rows.at[pl.ds(base + g * G, G)],
                             idxr_v.at[slot], isem)
            pltpu.async_copy(cols.at[pl.ds(base + g * G, G)],
                             idxc_v.at[slot], isem)

        def _stage_wait(slot):
            pltpu.make_async_copy(rows.at[pl.ds(base, G)], idxr_v.at[slot],
                                  isem).wait()
            pltpu.make_async_copy(cols.at[pl.ds(base, G)], idxc_v.at[slot],
                                  isem).wait()

        def _zfill(i, carry):
            zbuf[i, pl.ds(0, 16)] = jnp.zeros((16,), _f32)
            zbuf[i, pl.ds(16, 16)] = jnp.zeros((16,), _f32)
            return carry

        lax.fori_loop(0, ZROWS, _zfill, 0)
        if with_cnt:
            for k in range(LANES // 16):
                ones_v[pl.ds(k * 16, 16)] = jnp.ones((16,), _f32)
            for k in range(ZROWS // 16):
                zrow[pl.ds(k * 16, 16)] = jnp.zeros((16,), _f32)

        _stage(0, 0)

        def _zacc(j, carry):
            pltpu.async_copy(zbuf, acc_sh.at[pl.ds(stripe0 + j * ZROWS, ZROWS)],
                             zsem)
            if with_cnt:
                pltpu.async_copy(zrow,
                                 cnt_sh.at[pl.ds(stripe0 + j * ZROWS, ZROWS)],
                                 zsem)
            return carry

        def _zdrain(j, carry):
            pltpu.make_async_copy(zbuf, acc_sh.at[pl.ds(stripe0, ZROWS)],
                                  zsem).wait()
            if with_cnt:
                pltpu.make_async_copy(zrow, cnt_sh.at[pl.ds(stripe0, ZROWS)],
                                      zsem).wait()
            return carry

        lax.fori_loop(0, STRIPE // ZROWS, _zacc, 0)
        lax.fori_loop(0, STRIPE // ZROWS, _zdrain, 0)
        plsc.subcore_barrier()

        def _gather(slot, j, buf):
            @pl.when(c == 0)
            def _():
                pltpu.async_copy(m_lo.at[idxr_v.at[slot, j]], val_v.at[buf],
                                 gsem[buf])

            @pl.when(c == 1)
            def _():
                pltpu.async_copy(m_hi.at[idxr_v.at[slot, j]], val_v.at[buf],
                                 gsem[buf])

        def _gwait(buf):
            pltpu.make_async_copy(m_lo.at[idxr_v.at[0, 0]], val_v.at[buf],
                                  gsem[buf]).wait()

        def _scatter(slot, j, buf):
            pltpu.async_copy(val_v.at[buf], acc_sh.at[idxc_v.at[slot, j]],
                             ssem[buf], add=True)

        def _swait(buf):
            pltpu.make_async_copy(val_v.at[buf], acc_sh.at[idxc_v.at[0, 0]],
                                  ssem[buf]).wait()

        def _cnt_active(g):
            return ((c == 0) & (g < NG // 2)) | ((c == 1) & (g >= NG // 2))

        def _count(g, slot, j):
            @pl.when(_cnt_active(g))
            def _():
                pltpu.async_copy(ones_v, cnt_sh.at[idxc_v.at[slot, j]], csem,
                                 add=True)

        def _group(g, carry):
            slot = jnp.bitwise_and(g, 1)
            _stage_wait(slot)

            @pl.when(g + 1 < NG)
            def _():
                _stage(g + 1, 1 - slot)

            for j in range(G):
                buf = j % R
                if j < R:
                    @pl.when(g > 0)
                    def _():
                        _swait(buf)
                else:
                    _swait(buf)
                _gather(slot, j, buf)
                if j >= 2:
                    _gwait((j - 2) % R)
                    _scatter(slot, j - 2, (j - 2) % R)
                    if with_cnt:
                        _count(g, slot, j - 2)
            for j in range(G - 2, G):
                _gwait(j % R)
                _scatter(slot, j, j % R)
                if with_cnt:
                    _count(g, slot, j)
            if with_cnt:
                @pl.when(_cnt_active(g))
                def _():
                    for _j in range(G):
                        pltpu.make_async_copy(
                            ones_v, cnt_sh.at[idxc_v.at[0, 0]], csem).wait()
            return carry

        lax.fori_loop(0, NG, _group, 0)
        for buf in range(R):
            _swait(buf)
        plsc.subcore_barrier()

        pltpu.sync_copy(acc_sh.at[pl.ds(stripe0, STRIPE)],
                        s_out.at[c, pl.ds(stripe0, STRIPE)])
        if with_cnt:
            pltpu.sync_copy(cnt_sh.at[pl.ds(stripe0, STRIPE)],
                            cnt_out.at[c, pl.ds(stripe0, STRIPE)])

    return pl.kernel(
        body, out_type=out_type, mesh=mesh, scratch_types=scratch,
        compiler_params=pltpu.CompilerParams(use_tc_tiling_on_sc=False))


def _dot(a, b):
    return jnp.dot(a, b, preferred_element_type=_f32)


def _t1_body(x_ref, wa, ba, wb, bb, lo_ref, hi_ref):
    h = jnp.maximum(_dot(x_ref[...], wa[...]) + ba[...], 0.0)
    m = _dot(h, wb[...]) + bb[...]
    lo_ref[...] = m[:, :HQ]
    hi_ref[...] = m[:, HQ:]


def _t2_body(x_ref, s0, s1, c0, c1, w1c0, w1c1, b1c, w1d, b1d,
             w2a0, w2a1, b2a, w2b, b2b, u_ref, lo_ref, hi_ref):
    cnt = jnp.maximum(c0[...] + c1[...], 1.0).reshape(-1, 1)
    agg = jnp.concatenate([s0[...], s1[...]], axis=1) / cnt
    t = _dot(x_ref[...], w1c0[...]) + _dot(agg, w1c1[...]) + b1c[...]
    u = _dot(jnp.maximum(t, 0.0), w1d[...]) + b1d[...]
    u_ref[...] = u
    t2 = _dot(x_ref[...], w2a0[...]) + _dot(u, w2a1[...]) + b2a[...]
    m2 = _dot(jnp.maximum(t2, 0.0), w2b[...]) + b2b[...]
    lo_ref[...] = m2[:, :HQ]
    hi_ref[...] = m2[:, HQ:]


def _t3_body(x_ref, u, s0, s1, c0, c1, w2c0, w2c1, w2c2, b2c, wd, bd,
             o_ref):
    cnt = jnp.maximum(c0[...] + c1[...], 1.0).reshape(-1, 1)
    agg = jnp.concatenate([s0[...], s1[...]], axis=1) / cnt
    t = (_dot(x_ref[...], w2c0[...]) + _dot(u[...], w2c1[...])
         + _dot(agg, w2c2[...]) + b2c[...])
    o_ref[...] = _dot(jnp.maximum(t, 0.0), wd[...]) + bd[...]


def _row_spec(width):
    if width is None:
        return pl.BlockSpec((BM,), lambda i: (i,))
    return pl.BlockSpec((BM, width), lambda i: (i, 0))


def _full_spec(shape):
    return pl.BlockSpec(shape, lambda i: (0,) * len(shape))


def _tc_call(body, in_widths, full_shapes, out_widths):
    grid = (pl.cdiv(N, BM),)
    out_shape = [jax.ShapeDtypeStruct((N, w), _f32) for w in out_widths]
    out_specs = [_row_spec(w) for w in out_widths]
    if len(out_widths) == 1:
        out_shape, out_specs = out_shape[0], out_specs[0]
    return pl.pallas_call(
        body,
        grid=grid,
        in_specs=[_row_spec(w) for w in in_widths] + [_full_spec(s) for s in full_shapes],
        out_specs=out_specs,
        out_shape=out_shape,
    )


def _edge_scatter(m_lo, m_hi, rows_p, cols_p, first):
    if first:
        sa, cnt = _make_sc_pass(True)(m_lo, m_hi, rows_p, cols_p)
        return (sa[0, :N], sa[1, :N]), (cnt[0], cnt[1])
    (sa,) = _make_sc_pass(False)(m_lo, m_hi, rows_p, cols_p)
    return (sa[0, :N], sa[1, :N]), None


def kernel(x, edge_index, edge_attr, W1a, b1a, W1b, b1b, W1c, b1c, W1d, b1d,
           W2a, b2a, W2b, b2b, W2c, b2c, W2d, b2d):
    del edge_attr
    row = edge_index[0].astype(jnp.int32)
    col = edge_index[1].astype(jnp.int32)
    pad = EP - E
    rows_p = jnp.concatenate([row, jnp.zeros((pad,), jnp.int32)]).reshape(EROWS, LANES)
    cols_p = jnp.concatenate([col, jnp.full((pad,), TRASH, jnp.int32)]).reshape(EROWS, LANES)

    r2 = lambda b: b.reshape(1, -1)

    m1_lo, m1_hi = _tc_call(
        _t1_body, [F_IN], [(F_IN, H), (1, H), (H, H), (1, H)], [HQ, HQ])(
        x, W1a, r2(b1a), W1b, r2(b1b))
    s1p, (c0, c1) = _edge_scatter(m1_lo, m1_hi, rows_p, cols_p, first=True)

    u, m2_lo, m2_hi = _tc_call(
        _t2_body, [F_IN, HQ, HQ, None, None],
        [(F_IN, H), (H, H), (1, H), (H, H), (1, H),
         (F_IN, H), (H, H), (1, H), (H, H), (1, H)],
        [H, HQ, HQ],
    )(x, *s1p, c0, c1,
      W1c[:F_IN], W1c[F_IN:], r2(b1c), W1d, r2(b1d),
      W2a[:F_IN], W2a[F_IN:], r2(b2a), W2b, r2(b2b))

    s2p, _ = _edge_scatter(m2_lo, m2_hi, rows_p, cols_p, first=False)

    out = _tc_call(
        _t3_body, [F_IN, H, HQ, HQ, None, None],
        [(F_IN, H), (H, H), (H, H), (1, H), (H, OUT), (1, OUT)],
        [OUT],
    )(x, u, *s2p, c0, c1,
      W2c[:F_IN], W2c[F_IN:F_IN + H], W2c[F_IN + H:], r2(b2c), W2d, r2(b2d))
    return out


# speedup vs baseline: 11.2550x; 1.0758x over previous
import functools

import jax
import jax.numpy as jnp
from jax import lax
from jax.experimental import pallas as pl
from jax.experimental.pallas import tpu as pltpu
from jax.experimental.pallas import tpu_sc as plsc

N = 50000
E = 800000
F_IN = 128
H = 64
OUT = 2
HQ = 32
NQ = H // HQ

NC = 2
NS = 16
LANES = 128
NCH = 392
G = 8
NG = NCH // G
R = 4
EP = NS * NCH * LANES
EROWS = EP // LANES
TRASH = N
ACC_ROWS = 50176
STRIPE = ACC_ROWS // NS
ZROWS = 32

BM = 2048

_f32 = jnp.float32


@functools.lru_cache(maxsize=None)
def _make_sc_pass(with_cnt: bool):
    mesh = plsc.VectorSubcoreMesh(core_axis_name="c", subcore_axis_name="s",
                                  num_cores=NC, num_subcores=NS)
    out_type = [jax.ShapeDtypeStruct((NC, ACC_ROWS, HQ), _f32)]
    scratch = [
        pltpu.VMEM((2, G, LANES), jnp.int32),
        pltpu.VMEM((2, G, LANES), jnp.int32),
        pltpu.VMEM((R, LANES, HQ), _f32),
        pltpu.VMEM((ZROWS, HQ), _f32),
        pltpu.MemorySpace.VMEM_SHARED((ACC_ROWS, HQ), _f32),
        [pltpu.SemaphoreType.DMA] * R,
        [pltpu.SemaphoreType.DMA] * R,
        pltpu.SemaphoreType.DMA,
        pltpu.SemaphoreType.DMA,
    ]
    if with_cnt:
        out_type.append(jax.ShapeDtypeStruct((NC, ACC_ROWS), _f32))
        scratch += [
            pltpu.VMEM((LANES,), _f32),
            pltpu.VMEM((ZROWS,), _f32),
            pltpu.MemorySpace.VMEM_SHARED((ACC_ROWS,), _f32),
            pltpu.SemaphoreType.DMA,
        ]

    def body(m_lo, m_hi, rows, cols, *refs):
        if with_cnt:
            (s_out, cnt_out, idxr_v, idxc_v, val_v, zbuf, acc_sh, gsem, ssem,
             isem, zsem, ones_v, zrow, cnt_sh, csem) = refs
        else:
            (s_out, idxr_v, idxc_v, val_v, zbuf, acc_sh, gsem, ssem,
             isem, zsem) = refs
        c = lax.axis_index("c")
        s = lax.axis_index("s")
        base = s * NCH
        stripe0 = s * STRIPE

        def _stage(g, slot):
            pltpu.async_copy(rows.at[pl.ds(base + g * G, G)],
                             idxr_v.at[slot], isem)
            pltpu.async_copy(cols.at[pl.ds(base + g * G, G)],
                             idxc_v.at[slot], isem)

        def _stage_wait(slot):
            pltpu.make_async_copy(rows.at[pl.ds(base, G)], idxr_v.at[slot],
                                  isem).wait()
            pltpu.make_async_copy(cols.at[pl.ds(base, G)], idxc_v.at[slot],
                                  isem).wait()

        def _zfill(i, carry):
            zbuf[i, pl.ds(0, 16)] = jnp.zeros((16,), _f32)
            zbuf[i, pl.ds(16, 16)] = jnp.zeros((16,), _f32)
            return carry

        lax.fori_loop(0, ZROWS, _zfill, 0)
        if with_cnt:
            for k in range(LANES // 16):
                ones_v[pl.ds(k * 16, 16)] = jnp.ones((16,), _f32)
            for k in range(ZROWS // 16):
                zrow[pl.ds(k * 16, 16)] = jnp.zeros((16,), _f32)

        _stage(0, 0)

        def _zacc(j, carry):
            pltpu.async_copy(zbuf, acc_sh.at[pl.ds(stripe0 + j * ZROWS, ZROWS)],
                             zsem)
            if with_cnt:
                pltpu.async_copy(zrow,
                                 cnt_sh.at[pl.ds(stripe0 + j * ZROWS, ZROWS)],
                                 zsem)
            return carry

        def _zdrain(j, carry):
            pltpu.make_async_copy(zbuf, acc_sh.at[pl.ds(stripe0, ZROWS)],
                                  zsem).wait()
            if with_cnt:
                pltpu.make_async_copy(zrow, cnt_sh.at[pl.ds(stripe0, ZROWS)],
                                      zsem).wait()
            return carry

        lax.fori_loop(0, STRIPE // ZROWS, _zacc, 0)
        lax.fori_loop(0, STRIPE // ZROWS, _zdrain, 0)
        plsc.subcore_barrier()

        def _gather(slot, j, buf):
            @pl.when(c == 0)
            def _():
                pltpu.async_copy(m_lo.at[idxr_v.at[slot, j]], val_v.at[buf],
                                 gsem[buf])

            @pl.when(c == 1)
            def _():
                pltpu.async_copy(m_hi.at[idxr_v.at[slot, j]], val_v.at[buf],
                                 gsem[buf])

        def _gwait(buf):
            pltpu.make_async_copy(m_lo.at[idxr_v.at[0, 0]], val_v.at[buf],
                                  gsem[buf]).wait()

        def _scatter(slot, j, buf):
            pltpu.async_copy(val_v.at[buf], acc_sh.at[idxc_v.at[slot, j]],
                             ssem[buf], add=True)

        def _swait(buf):
            pltpu.make_async_copy(val_v.at[buf], acc_sh.at[idxc_v.at[0, 0]],
                                  ssem[buf]).wait()

        def _cnt_active(g):
            return ((c == 0) & (g < NG // 2)) | ((c == 1) & (g >= NG // 2))

        def _count(g, slot, j):
            @pl.when(_cnt_active(g))
            def _():
                pltpu.async_copy(ones_v, cnt_sh.at[idxc_v.at[slot, j]], csem,
                                 add=True)

        def _group(g, carry):
            slot = jnp.bitwise_and(g, 1)
            _stage_wait(slot)

            @pl.when(g + 1 < NG)
            def _():
                _stage(g + 1, 1 - slot)

            for j in range(G):
                buf = j % R
                if j < R:
                    @pl.when(g > 0)
                    def _():
                        _swait(buf)
                else:
                    _swait(buf)
                _gather(slot, j, buf)
                if j >= 2:
                    _gwait((j - 2) % R)
                    _scatter(slot, j - 2, (j - 2) % R)
                    if with_cnt:
                        _count(g, slot, j - 2)
            for j in range(G - 2, G):
                _gwait(j % R)
                _scatter(slot, j, j % R)
                if with_cnt:
                    _count(g, slot, j)
            if with_cnt:
                @pl.when(_cnt_active(g))
                def _():
                    for _j in range(G):
                        pltpu.make_async_copy(
                            ones_v, cnt_sh.at[idxc_v.at[0, 0]], csem).wait()
            return carry

        lax.fori_loop(0, NG, _group, 0)
        for buf in range(R):
            _swait(buf)
        plsc.subcore_barrier()

        pltpu.sync_copy(acc_sh.at[pl.ds(stripe0, STRIPE)],
                        s_out.at[c, pl.ds(stripe0, STRIPE)])
        if with_cnt:
            pltpu.sync_copy(cnt_sh.at[pl.ds(stripe0, STRIPE)],
                            cnt_out.at[c, pl.ds(stripe0, STRIPE)])

    return pl.kernel(
        body, out_type=out_type, mesh=mesh, scratch_types=scratch,
        compiler_params=pltpu.CompilerParams(use_tc_tiling_on_sc=False))


def _dot(a, b):
    return jnp.dot(a, b, preferred_element_type=_f32)


def _t1_body(x_ref, wa, ba, wb, bb, lo_ref, hi_ref):
    h = jnp.maximum(_dot(x_ref[...], wa[...]) + ba[...], 0.0)
    m = _dot(h, wb[...]) + bb[...]
    lo_ref[...] = m[:, :HQ]
    hi_ref[...] = m[:, HQ:]


def _t2_body(x_ref, sp, c0, c1, w1c0, w1c1, b1c, w1d, b1d,
             w2a0, w2a1, b2a, w2b, b2b, u_ref, lo_ref, hi_ref):
    cnt = jnp.maximum(c0[...] + c1[...], 1.0).reshape(-1, 1)
    s = sp[...]
    agg = jnp.concatenate([s[0], s[1]], axis=1) / cnt
    t = _dot(x_ref[...], w1c0[...]) + _dot(agg, w1c1[...]) + b1c[...]
    u = _dot(jnp.maximum(t, 0.0), w1d[...]) + b1d[...]
    u_ref[...] = u
    t2 = _dot(x_ref[...], w2a0[...]) + _dot(u, w2a1[...]) + b2a[...]
    m2 = _dot(jnp.maximum(t2, 0.0), w2b[...]) + b2b[...]
    lo_ref[...] = m2[:, :HQ]
    hi_ref[...] = m2[:, HQ:]


def _t3_body(x_ref, u, sp, c0, c1, w2c0, w2c1, w2c2, b2c, wd, bd,
             o_ref):
    cnt = jnp.maximum(c0[...] + c1[...], 1.0).reshape(-1, 1)
    s = sp[...]
    agg = jnp.concatenate([s[0], s[1]], axis=1) / cnt
    t = (_dot(x_ref[...], w2c0[...]) + _dot(u[...], w2c1[...])
         + _dot(agg, w2c2[...]) + b2c[...])
    o_ref[...] = _dot(jnp.maximum(t, 0.0), wd[...]) + bd[...]


def _row_spec(width):
    if width is None:
        return pl.BlockSpec((BM,), lambda i: (i,))
    if width == "sp":
        return pl.BlockSpec((2, BM, HQ), lambda i: (0, i, 0))
    return pl.BlockSpec((BM, width), lambda i: (i, 0))


def _full_spec(shape):
    return pl.BlockSpec(shape, lambda i: (0,) * len(shape))


def _tc_call(body, in_widths, full_shapes, out_widths):
    grid = (pl.cdiv(N, BM),)
    out_shape = [jax.ShapeDtypeStruct((N, w), _f32) for w in out_widths]
    out_specs = [_row_spec(w) for w in out_widths]
    if len(out_widths) == 1:
        out_shape, out_specs = out_shape[0], out_specs[0]
    return pl.pallas_call(
        body,
        grid=grid,
        in_specs=[_row_spec(w) for w in in_widths] + [_full_spec(s) for s in full_shapes],
        out_specs=out_specs,
        out_shape=out_shape,
    )


def _edge_scatter(m_lo, m_hi, rows_p, cols_p, first):
    if first:
        sa, cnt = _make_sc_pass(True)(m_lo, m_hi, rows_p, cols_p)
        return sa, (cnt[0], cnt[1])
    (sa,) = _make_sc_pass(False)(m_lo, m_hi, rows_p, cols_p)
    return sa, None


def kernel(x, edge_index, edge_attr, W1a, b1a, W1b, b1b, W1c, b1c, W1d, b1d,
           W2a, b2a, W2b, b2b, W2c, b2c, W2d, b2d):
    del edge_attr
    row = edge_index[0].astype(jnp.int32)
    col = edge_index[1].astype(jnp.int32)
    pad = EP - E
    rows_p = jnp.concatenate([row, jnp.zeros((pad,), jnp.int32)]).reshape(EROWS, LANES)
    cols_p = jnp.concatenate([col, jnp.full((pad,), TRASH, jnp.int32)]).reshape(EROWS, LANES)

    r2 = lambda b: b.reshape(1, -1)

    m1_lo, m1_hi = _tc_call(
        _t1_body, [F_IN], [(F_IN, H), (1, H), (H, H), (1, H)], [HQ, HQ])(
        x, W1a, r2(b1a), W1b, r2(b1b))
    s1p, (c0, c1) = _edge_scatter(m1_lo, m1_hi, rows_p, cols_p, first=True)

    u, m2_lo, m2_hi = _tc_call(
        _t2_body, [F_IN, "sp", None, None],
        [(F_IN, H), (H, H), (1, H), (H, H), (1, H),
         (F_IN, H), (H, H), (1, H), (H, H), (1, H)],
        [H, HQ, HQ],
    )(x, s1p, c0, c1,
      W1c[:F_IN], W1c[F_IN:], r2(b1c), W1d, r2(b1d),
      W2a[:F_IN], W2a[F_IN:], r2(b2a), W2b, r2(b2b))

    s2p, _ = _edge_scatter(m2_lo, m2_hi, rows_p, cols_p, first=False)

    out = _tc_call(
        _t3_body, [F_IN, H, "sp", None, None],
        [(F_IN, H), (H, H), (H, H), (1, H), (H, OUT), (1, OUT)],
        [OUT],
    )(x, u, s2p, c0, c1,
      W2c[:F_IN], W2c[F_IN:F_IN + H], W2c[F_IN + H:], r2(b2c), W2d, r2(b2d))
    return out


# speedup vs baseline: 11.5933x; 1.0301x over previous
import functools

import jax
import jax.numpy as jnp
from jax import lax
from jax.experimental import pallas as pl
from jax.experimental.pallas import tpu as pltpu
from jax.experimental.pallas import tpu_sc as plsc

N = 50000
E = 800000
F_IN = 128
H = 64
OUT = 2
HQ = 32
NQ = H // HQ

NC = 2
NS = 16
LANES = 128
NCH = 392
G = 8
NG = NCH // G
R = 4
EP = NS * NCH * LANES
EROWS = EP // LANES
TRASH = N
ACC_ROWS = 50176
STRIPE = ACC_ROWS // NS
ZROWS = 32

BM = 4096

_f32 = jnp.float32


@functools.lru_cache(maxsize=None)
def _make_sc_pass(with_cnt: bool):
    mesh = plsc.VectorSubcoreMesh(core_axis_name="c", subcore_axis_name="s",
                                  num_cores=NC, num_subcores=NS)
    out_type = [jax.ShapeDtypeStruct((NC, ACC_ROWS, HQ), _f32)]
    scratch = [
        pltpu.VMEM((2, G, LANES), jnp.int32),
        pltpu.VMEM((2, G, LANES), jnp.int32),
        pltpu.VMEM((R, LANES, HQ), _f32),
        pltpu.VMEM((ZROWS, HQ), _f32),
        pltpu.MemorySpace.VMEM_SHARED((ACC_ROWS, HQ), _f32),
        [pltpu.SemaphoreType.DMA] * R,
        [pltpu.SemaphoreType.DMA] * R,
        pltpu.SemaphoreType.DMA,
        pltpu.SemaphoreType.DMA,
    ]
    if with_cnt:
        out_type.append(jax.ShapeDtypeStruct((NC, ACC_ROWS), _f32))
        scratch += [
            pltpu.VMEM((LANES,), _f32),
            pltpu.VMEM((ZROWS,), _f32),
            pltpu.MemorySpace.VMEM_SHARED((ACC_ROWS,), _f32),
            pltpu.SemaphoreType.DMA,
        ]

    def body(m_lo, m_hi, rows, cols, *refs):
        if with_cnt:
            (s_out, cnt_out, idxr_v, idxc_v, val_v, zbuf, acc_sh, gsem, ssem,
             isem, zsem, ones_v, zrow, cnt_sh, csem) = refs
        else:
            (s_out, idxr_v, idxc_v, val_v, zbuf, acc_sh, gsem, ssem,
             isem, zsem) = refs
        c = lax.axis_index("c")
        s = lax.axis_index("s")
        base = s * NCH
        stripe0 = s * STRIPE

        def _stage(g, slot):
            pltpu.async_copy(rows.at[pl.ds(base + g * G, G)],
                             idxr_v.at[slot], isem)
            pltpu.async_copy(cols.at[pl.ds(base + g * G, G)],
                             idxc_v.at[slot], isem)

        def _stage_wait(slot):
            pltpu.make_async_copy(rows.at[pl.ds(base, G)], idxr_v.at[slot],
                                  isem).wait()
            pltpu.make_async_copy(cols.at[pl.ds(base, G)], idxc_v.at[slot],
                                  isem).wait()

        def _zfill(i, carry):
            zbuf[i, pl.ds(0, 16)] = jnp.zeros((16,), _f32)
            zbuf[i, pl.ds(16, 16)] = jnp.zeros((16,), _f32)
            return carry

        lax.fori_loop(0, ZROWS, _zfill, 0)
        if with_cnt:
            for k in range(LANES // 16):
                ones_v[pl.ds(k * 16, 16)] = jnp.ones((16,), _f32)
            for k in range(ZROWS // 16):
                zrow[pl.ds(k * 16, 16)] = jnp.zeros((16,), _f32)

        _stage(0, 0)

        def _zacc(j, carry):
            pltpu.async_copy(zbuf, acc_sh.at[pl.ds(stripe0 + j * ZROWS, ZROWS)],
                             zsem)
            if with_cnt:
                pltpu.async_copy(zrow,
                                 cnt_sh.at[pl.ds(stripe0 + j * ZROWS, ZROWS)],
                                 zsem)
            return carry

        def _zdrain(j, carry):
            pltpu.make_async_copy(zbuf, acc_sh.at[pl.ds(stripe0, ZROWS)],
                                  zsem).wait()
            if with_cnt:
                pltpu.make_async_copy(zrow, cnt_sh.at[pl.ds(stripe0, ZROWS)],
                                      zsem).wait()
            return carry

        lax.fori_loop(0, STRIPE // ZROWS, _zacc, 0)
        lax.fori_loop(0, STRIPE // ZROWS, _zdrain, 0)
        plsc.subcore_barrier()

        def _gather(slot, j, buf):
            @pl.when(c == 0)
            def _():
                pltpu.async_copy(m_lo.at[idxr_v.at[slot, j]], val_v.at[buf],
                                 gsem[buf])

            @pl.when(c == 1)
            def _():
                pltpu.async_copy(m_hi.at[idxr_v.at[slot, j]], val_v.at[buf],
                                 gsem[buf])

        def _gwait(buf):
            pltpu.make_async_copy(m_lo.at[idxr_v.at[0, 0]], val_v.at[buf],
                                  gsem[buf]).wait()

        def _scatter(slot, j, buf):
            pltpu.async_copy(val_v.at[buf], acc_sh.at[idxc_v.at[slot, j]],
                             ssem[buf], add=True)

        def _swait(buf):
            pltpu.make_async_copy(val_v.at[buf], acc_sh.at[idxc_v.at[0, 0]],
                                  ssem[buf]).wait()

        def _cnt_active(g):
            return ((c == 0) & (g < NG // 2)) | ((c == 1) & (g >= NG // 2))

        def _count(g, slot, j):
            @pl.when(_cnt_active(g))
            def _():
                pltpu.async_copy(ones_v, cnt_sh.at[idxc_v.at[slot, j]], csem,
                                 add=True)

        def _group(g, carry):
            slot = jnp.bitwise_and(g, 1)
            _stage_wait(slot)

            @pl.when(g + 1 < NG)
            def _():
                _stage(g + 1, 1 - slot)

            for j in range(G):
                buf = j % R
                if j < R:
                    @pl.when(g > 0)
                    def _():
                        _swait(buf)
                else:
                    _swait(buf)
                _gather(slot, j, buf)
                if j >= 2:
                    _gwait((j - 2) % R)
                    _scatter(slot, j - 2, (j - 2) % R)
                    if with_cnt:
                        _count(g, slot, j - 2)
            for j in range(G - 2, G):
                _gwait(j % R)
                _scatter(slot, j, j % R)
                if with_cnt:
                    _count(g, slot, j)
            if with_cnt:
                @pl.when(_cnt_active(g))
                def _():
                    for _j in range(G):
                        pltpu.make_async_copy(
                            ones_v, cnt_sh.at[idxc_v.at[0, 0]], csem).wait()
            return carry

        lax.fori_loop(0, NG, _group, 0)
        for buf in range(R):
            _swait(buf)
        plsc.subcore_barrier()

        pltpu.sync_copy(acc_sh.at[pl.ds(stripe0, STRIPE)],
                        s_out.at[c, pl.ds(stripe0, STRIPE)])
        if with_cnt:
            pltpu.sync_copy(cnt_sh.at[pl.ds(stripe0, STRIPE)],
                            cnt_out.at[c, pl.ds(stripe0, STRIPE)])

    return pl.kernel(
        body, out_type=out_type, mesh=mesh, scratch_types=scratch,
        compiler_params=pltpu.CompilerParams(use_tc_tiling_on_sc=False))


def _dot(a, b):
    return jnp.dot(a, b, preferred_element_type=_f32)


def _t1_body(x_ref, wa, ba, wb, bb, lo_ref, hi_ref):
    h = jnp.maximum(_dot(x_ref[...], wa[...]) + ba[...], 0.0)
    m = _dot(h, wb[...]) + bb[...]
    lo_ref[...] = m[:, :HQ]
    hi_ref[...] = m[:, HQ:]


def _t2_body(x_ref, sp, c0, c1, w1c0, w1c1, b1c, w1d, b1d,
             w2a0, w2a1, b2a, w2b, b2b, u_ref, lo_ref, hi_ref):
    cnt = jnp.maximum(c0[...] + c1[...], 1.0).reshape(-1, 1)
    s = sp[...]
    agg = jnp.concatenate([s[0], s[1]], axis=1) / cnt
    t = _dot(x_ref[...], w1c0[...]) + _dot(agg, w1c1[...]) + b1c[...]
    u = _dot(jnp.maximum(t, 0.0), w1d[...]) + b1d[...]
    u_ref[...] = u
    t2 = _dot(x_ref[...], w2a0[...]) + _dot(u, w2a1[...]) + b2a[...]
    m2 = _dot(jnp.maximum(t2, 0.0), w2b[...]) + b2b[...]
    lo_ref[...] = m2[:, :HQ]
    hi_ref[...] = m2[:, HQ:]


def _t3_body(x_ref, u, sp, c0, c1, w2c0, w2c1, w2c2, b2c, wd, bd,
             o_ref):
    cnt = jnp.maximum(c0[...] + c1[...], 1.0).reshape(-1, 1)
    s = sp[...]
    agg = jnp.concatenate([s[0], s[1]], axis=1) / cnt
    t = (_dot(x_ref[...], w2c0[...]) + _dot(u[...], w2c1[...])
         + _dot(agg, w2c2[...]) + b2c[...])
    o_ref[...] = _dot(jnp.maximum(t, 0.0), wd[...]) + bd[...]


def _row_spec(width):
    if width is None:
        return pl.BlockSpec((BM,), lambda i: (i,))
    if width == "sp":
        return pl.BlockSpec((2, BM, HQ), lambda i: (0, i, 0))
    return pl.BlockSpec((BM, width), lambda i: (i, 0))


def _full_spec(shape):
    return pl.BlockSpec(shape, lambda i: (0,) * len(shape))


def _tc_call(body, in_widths, full_shapes, out_widths):
    grid = (pl.cdiv(N, BM),)
    out_shape = [jax.ShapeDtypeStruct((N, w), _f32) for w in out_widths]
    out_specs = [_row_spec(w) for w in out_widths]
    if len(out_widths) == 1:
        out_shape, out_specs = out_shape[0], out_specs[0]
    return pl.pallas_call(
        body,
        grid=grid,
        in_specs=[_row_spec(w) for w in in_widths] + [_full_spec(s) for s in full_shapes],
        out_specs=out_specs,
        out_shape=out_shape,
    )


def _edge_scatter(m_lo, m_hi, rows_p, cols_p, first):
    if first:
        sa, cnt = _make_sc_pass(True)(m_lo, m_hi, rows_p, cols_p)
        return sa, (cnt[0], cnt[1])
    (sa,) = _make_sc_pass(False)(m_lo, m_hi, rows_p, cols_p)
    return sa, None


def kernel(x, edge_index, edge_attr, W1a, b1a, W1b, b1b, W1c, b1c, W1d, b1d,
           W2a, b2a, W2b, b2b, W2c, b2c, W2d, b2d):
    del edge_attr
    row = edge_index[0].astype(jnp.int32)
    col = edge_index[1].astype(jnp.int32)
    pad = EP - E
    rows_p = jnp.concatenate([row, jnp.zeros((pad,), jnp.int32)]).reshape(EROWS, LANES)
    cols_p = jnp.concatenate([col, jnp.full((pad,), TRASH, jnp.int32)]).reshape(EROWS, LANES)

    r2 = lambda b: b.reshape(1, -1)

    m1_lo, m1_hi = _tc_call(
        _t1_body, [F_IN], [(F_IN, H), (1, H), (H, H), (1, H)], [HQ, HQ])(
        x, W1a, r2(b1a), W1b, r2(b1b))
    s1p, (c0, c1) = _edge_scatter(m1_lo, m1_hi, rows_p, cols_p, first=True)

    u, m2_lo, m2_hi = _tc_call(
        _t2_body, [F_IN, "sp", None, None],
        [(F_IN, H), (H, H), (1, H), (H, H), (1, H),
         (F_IN, H), (H, H), (1, H), (H, H), (1, H)],
        [H, HQ, HQ],
    )(x, s1p, c0, c1,
      W1c[:F_IN], W1c[F_IN:], r2(b1c), W1d, r2(b1d),
      W2a[:F_IN], W2a[F_IN:], r2(b2a), W2b, r2(b2b))

    s2p, _ = _edge_scatter(m2_lo, m2_hi, rows_p, cols_p, first=False)

    out = _tc_call(
        _t3_body, [F_IN, H, "sp", None, None],
        [(F_IN, H), (H, H), (H, H), (1, H), (H, OUT), (1, OUT)],
        [OUT],
    )(x, u, s2p, c0, c1,
      W2c[:F_IN], W2c[F_IN:F_IN + H], W2c[F_IN + H:], r2(b2c), W2d, r2(b2d))
    return out


# speedup vs baseline: 11.6377x; 1.0038x over previous
import functools

import jax
import jax.numpy as jnp
from jax import lax
from jax.experimental import pallas as pl
from jax.experimental.pallas import tpu as pltpu
from jax.experimental.pallas import tpu_sc as plsc

N = 50000
E = 800000
F_IN = 128
H = 64
OUT = 2
HQ = 32
NQ = H // HQ

NC = 2
NS = 16
LANES = 128
NCH = 392
G = 8
NG = NCH // G
R = 4
EP = NS * NCH * LANES
EROWS = EP // LANES
TRASH = N
ACC_ROWS = 50176
STRIPE = ACC_ROWS // NS
ZROWS = 32

BM = 8192

_f32 = jnp.float32


@functools.lru_cache(maxsize=None)
def _make_sc_pass(with_cnt: bool):
    mesh = plsc.VectorSubcoreMesh(core_axis_name="c", subcore_axis_name="s",
                                  num_cores=NC, num_subcores=NS)
    out_type = [jax.ShapeDtypeStruct((NC, ACC_ROWS, HQ), _f32)]
    scratch = [
        pltpu.VMEM((2, G, LANES), jnp.int32),
        pltpu.VMEM((2, G, LANES), jnp.int32),
        pltpu.VMEM((R, LANES, HQ), _f32),
        pltpu.VMEM((ZROWS, HQ), _f32),
        pltpu.MemorySpace.VMEM_SHARED((ACC_ROWS, HQ), _f32),
        [pltpu.SemaphoreType.DMA] * R,
        [pltpu.SemaphoreType.DMA] * R,
        pltpu.SemaphoreType.DMA,
        pltpu.SemaphoreType.DMA,
    ]
    if with_cnt:
        out_type.append(jax.ShapeDtypeStruct((NC, ACC_ROWS), _f32))
        scratch += [
            pltpu.VMEM((LANES,), _f32),
            pltpu.VMEM((ZROWS,), _f32),
            pltpu.MemorySpace.VMEM_SHARED((ACC_ROWS,), _f32),
            pltpu.SemaphoreType.DMA,
        ]

    def body(m_lo, m_hi, rows, cols, *refs):
        if with_cnt:
            (s_out, cnt_out, idxr_v, idxc_v, val_v, zbuf, acc_sh, gsem, ssem,
             isem, zsem, ones_v, zrow, cnt_sh, csem) = refs
        else:
            (s_out, idxr_v, idxc_v, val_v, zbuf, acc_sh, gsem, ssem,
             isem, zsem) = refs
        c = lax.axis_index("c")
        s = lax.axis_index("s")
        base = s * NCH
        stripe0 = s * STRIPE

        def _stage(g, slot):
            pltpu.async_copy(rows.at[pl.ds(base + g * G, G)],
                             idxr_v.at[slot], isem)
            pltpu.async_copy(cols.at[pl.ds(base + g * G, G)],
                             idxc_v.at[slot], isem)

        def _stage_wait(slot):
            pltpu.make_async_copy(rows.at[pl.ds(base, G)], idxr_v.at[slot],
                                  isem).wait()
            pltpu.make_async_copy(cols.at[pl.ds(base, G)], idxc_v.at[slot],
                                  isem).wait()

        def _zfill(i, carry):
            zbuf[i, pl.ds(0, 16)] = jnp.zeros((16,), _f32)
            zbuf[i, pl.ds(16, 16)] = jnp.zeros((16,), _f32)
            return carry

        lax.fori_loop(0, ZROWS, _zfill, 0)
        if with_cnt:
            for k in range(LANES // 16):
                ones_v[pl.ds(k * 16, 16)] = jnp.ones((16,), _f32)
            for k in range(ZROWS // 16):
                zrow[pl.ds(k * 16, 16)] = jnp.zeros((16,), _f32)

        _stage(0, 0)

        def _zacc(j, carry):
            pltpu.async_copy(zbuf, acc_sh.at[pl.ds(stripe0 + j * ZROWS, ZROWS)],
                             zsem)
            if with_cnt:
                pltpu.async_copy(zrow,
                                 cnt_sh.at[pl.ds(stripe0 + j * ZROWS, ZROWS)],
                                 zsem)
            return carry

        def _zdrain(j, carry):
            pltpu.make_async_copy(zbuf, acc_sh.at[pl.ds(stripe0, ZROWS)],
                                  zsem).wait()
            if with_cnt:
                pltpu.make_async_copy(zrow, cnt_sh.at[pl.ds(stripe0, ZROWS)],
                                      zsem).wait()
            return carry

        lax.fori_loop(0, STRIPE // ZROWS, _zacc, 0)
        lax.fori_loop(0, STRIPE // ZROWS, _zdrain, 0)
        plsc.subcore_barrier()

        def _gather(slot, j, buf):
            @pl.when(c == 0)
            def _():
                pltpu.async_copy(m_lo.at[idxr_v.at[slot, j]], val_v.at[buf],
                                 gsem[buf])

            @pl.when(c == 1)
            def _():
                pltpu.async_copy(m_hi.at[idxr_v.at[slot, j]], val_v.at[buf],
                                 gsem[buf])

        def _gwait(buf):
            pltpu.make_async_copy(m_lo.at[idxr_v.at[0, 0]], val_v.at[buf],
                                  gsem[buf]).wait()

        def _scatter(slot, j, buf):
            pltpu.async_copy(val_v.at[buf], acc_sh.at[idxc_v.at[slot, j]],
                             ssem[buf], add=True)

        def _swait(buf):
            pltpu.make_async_copy(val_v.at[buf], acc_sh.at[idxc_v.at[0, 0]],
                                  ssem[buf]).wait()

        def _cnt_active(g):
            return ((c == 0) & (g < NG // 2)) | ((c == 1) & (g >= NG // 2))

        def _count(g, slot, j):
            @pl.when(_cnt_active(g))
            def _():
                pltpu.async_copy(ones_v, cnt_sh.at[idxc_v.at[slot, j]], csem,
                                 add=True)

        def _group(g, carry):
            slot = jnp.bitwise_and(g, 1)
            _stage_wait(slot)

            @pl.when(g + 1 < NG)
            def _():
                _stage(g + 1, 1 - slot)

            for j in range(G):
                buf = j % R
                if j < R:
                    @pl.when(g > 0)
                    def _():
                        _swait(buf)
                else:
                    _swait(buf)
                _gather(slot, j, buf)
                if j >= 2:
                    _gwait((j - 2) % R)
                    _scatter(slot, j - 2, (j - 2) % R)
                    if with_cnt:
                        _count(g, slot, j - 2)
            for j in range(G - 2, G):
                _gwait(j % R)
                _scatter(slot, j, j % R)
                if with_cnt:
                    _count(g, slot, j)
            if with_cnt:
                @pl.when(_cnt_active(g))
                def _():
                    for _j in range(G):
                        pltpu.make_async_copy(
                            ones_v, cnt_sh.at[idxc_v.at[0, 0]], csem).wait()
            return carry

        lax.fori_loop(0, NG, _group, 0)
        for buf in range(R):
            _swait(buf)
        plsc.subcore_barrier()

        pltpu.sync_copy(acc_sh.at[pl.ds(stripe0, STRIPE)],
                        s_out.at[c, pl.ds(stripe0, STRIPE)])
        if with_cnt:
            pltpu.sync_copy(cnt_sh.at[pl.ds(stripe0, STRIPE)],
                            cnt_out.at[c, pl.ds(stripe0, STRIPE)])

    return pl.kernel(
        body, out_type=out_type, mesh=mesh, scratch_types=scratch,
        compiler_params=pltpu.CompilerParams(use_tc_tiling_on_sc=False))


def _dot(a, b):
    return jnp.dot(a, b, preferred_element_type=_f32)


def _t1_body(x_ref, wa, ba, wb, bb, lo_ref, hi_ref):
    h = jnp.maximum(_dot(x_ref[...], wa[...]) + ba[...], 0.0)
    m = _dot(h, wb[...]) + bb[...]
    lo_ref[...] = m[:, :HQ]
    hi_ref[...] = m[:, HQ:]


def _t2_body(x_ref, sp, c0, c1, w1c0, w1c1, b1c, w1d, b1d,
             w2a0, w2a1, b2a, w2b, b2b, u_ref, lo_ref, hi_ref):
    cnt = jnp.maximum(c0[...] + c1[...], 1.0).reshape(-1, 1)
    s = sp[...]
    agg = jnp.concatenate([s[0], s[1]], axis=1) / cnt
    t = _dot(x_ref[...], w1c0[...]) + _dot(agg, w1c1[...]) + b1c[...]
    u = _dot(jnp.maximum(t, 0.0), w1d[...]) + b1d[...]
    u_ref[...] = u
    t2 = _dot(x_ref[...], w2a0[...]) + _dot(u, w2a1[...]) + b2a[...]
    m2 = _dot(jnp.maximum(t2, 0.0), w2b[...]) + b2b[...]
    lo_ref[...] = m2[:, :HQ]
    hi_ref[...] = m2[:, HQ:]


def _t3_body(x_ref, u, sp, c0, c1, w2c0, w2c1, w2c2, b2c, wd, bd,
             o_ref):
    cnt = jnp.maximum(c0[...] + c1[...], 1.0).reshape(-1, 1)
    s = sp[...]
    agg = jnp.concatenate([s[0], s[1]], axis=1) / cnt
    t = (_dot(x_ref[...], w2c0[...]) + _dot(u[...], w2c1[...])
         + _dot(agg, w2c2[...]) + b2c[...])
    o_ref[...] = _dot(jnp.maximum(t, 0.0), wd[...]) + bd[...]


def _row_spec(width):
    if width is None:
        return pl.BlockSpec((BM,), lambda i: (i,))
    if width == "sp":
        return pl.BlockSpec((2, BM, HQ), lambda i: (0, i, 0))
    return pl.BlockSpec((BM, width), lambda i: (i, 0))


def _full_spec(shape):
    return pl.BlockSpec(shape, lambda i: (0,) * len(shape))


def _tc_call(body, in_widths, full_shapes, out_widths):
    grid = (pl.cdiv(N, BM),)
    out_shape = [jax.ShapeDtypeStruct((N, w), _f32) for w in out_widths]
    out_specs = [_row_spec(w) for w in out_widths]
    if len(out_widths) == 1:
        out_shape, out_specs = out_shape[0], out_specs[0]
    return pl.pallas_call(
        body,
        grid=grid,
        in_specs=[_row_spec(w) for w in in_widths] + [_full_spec(s) for s in full_shapes],
        out_specs=out_specs,
        out_shape=out_shape,
    )


def _edge_scatter(m_lo, m_hi, rows_p, cols_p, first):
    if first:
        sa, cnt = _make_sc_pass(True)(m_lo, m_hi, rows_p, cols_p)
        return sa, (cnt[0], cnt[1])
    (sa,) = _make_sc_pass(False)(m_lo, m_hi, rows_p, cols_p)
    return sa, None


def kernel(x, edge_index, edge_attr, W1a, b1a, W1b, b1b, W1c, b1c, W1d, b1d,
           W2a, b2a, W2b, b2b, W2c, b2c, W2d, b2d):
    del edge_attr
    row = edge_index[0].astype(jnp.int32)
    col = edge_index[1].astype(jnp.int32)
    pad = EP - E
    rows_p = jnp.concatenate([row, jnp.zeros((pad,), jnp.int32)]).reshape(EROWS, LANES)
    cols_p = jnp.concatenate([col, jnp.full((pad,), TRASH, jnp.int32)]).reshape(EROWS, LANES)

    r2 = lambda b: b.reshape(1, -1)

    m1_lo, m1_hi = _tc_call(
        _t1_body, [F_IN], [(F_IN, H), (1, H), (H, H), (1, H)], [HQ, HQ])(
        x, W1a, r2(b1a), W1b, r2(b1b))
    s1p, (c0, c1) = _edge_scatter(m1_lo, m1_hi, rows_p, cols_p, first=True)

    u, m2_lo, m2_hi = _tc_call(
        _t2_body, [F_IN, "sp", None, None],
        [(F_IN, H), (H, H), (1, H), (H, H), (1, H),
         (F_IN, H), (H, H), (1, H), (H, H), (1, H)],
        [H, HQ, HQ],
    )(x, s1p, c0, c1,
      W1c[:F_IN], W1c[F_IN:], r2(b1c), W1d, r2(b1d),
      W2a[:F_IN], W2a[F_IN:], r2(b2a), W2b, r2(b2b))

    s2p, _ = _edge_scatter(m2_lo, m2_hi, rows_p, cols_p, first=False)

    out = _tc_call(
        _t3_body, [F_IN, H, "sp", None, None],
        [(F_IN, H), (H, H), (H, H), (1, H), (H, OUT), (1, OUT)],
        [OUT],
    )(x, u, s2p, c0, c1,
      W2c[:F_IN], W2c[F_IN:F_IN + H], W2c[F_IN + H:], r2(b2c), W2d, r2(b2d))
    return out


# speedup vs baseline: 12.2565x; 1.0532x over previous
import functools

import jax
import jax.numpy as jnp
from jax import lax
from jax.experimental import pallas as pl
from jax.experimental.pallas import tpu as pltpu
from jax.experimental.pallas import tpu_sc as plsc

N = 50000
E = 800000
F_IN = 128
H = 64
OUT = 2
HQ = 32
NQ = H // HQ

NC = 2
NS = 16
LANES = 128
NCH = 392
G = 8
NG = NCH // G
R = 4
EP = NS * NCH * LANES
EROWS = EP // LANES
TRASH = N
ACC_ROWS = 50176
STRIPE = ACC_ROWS // NS
ZROWS = 32

BM = 8192

_f32 = jnp.float32


@functools.lru_cache(maxsize=None)
def _make_sc_pass(with_cnt: bool):
    mesh = plsc.VectorSubcoreMesh(core_axis_name="c", subcore_axis_name="s",
                                  num_cores=NC, num_subcores=NS)
    out_type = [jax.ShapeDtypeStruct((NC, ACC_ROWS, HQ), _f32)]
    scratch = [
        pltpu.VMEM((2, G, LANES), jnp.int32),
        pltpu.VMEM((2, G, LANES), jnp.int32),
        pltpu.VMEM((R, LANES, HQ), _f32),
        pltpu.VMEM((ZROWS, HQ), _f32),
        pltpu.MemorySpace.VMEM_SHARED((ACC_ROWS, HQ), _f32),
        [pltpu.SemaphoreType.DMA] * R,
        [pltpu.SemaphoreType.DMA] * R,
        pltpu.SemaphoreType.DMA,
        pltpu.SemaphoreType.DMA,
    ]
    if with_cnt:
        out_type.append(jax.ShapeDtypeStruct((NC, ACC_ROWS), _f32))
        scratch += [
            pltpu.VMEM((LANES,), _f32),
            pltpu.VMEM((ZROWS,), _f32),
            pltpu.MemorySpace.VMEM_SHARED((ACC_ROWS,), _f32),
            pltpu.SemaphoreType.DMA,
        ]

    def body(m_lo, m_hi, rc, *refs):
        if with_cnt:
            (s_out, cnt_out, idxr_v, idxc_v, val_v, zbuf, acc_sh, gsem, ssem,
             isem, zsem, ones_v, zrow, cnt_sh, csem) = refs
        else:
            (s_out, idxr_v, idxc_v, val_v, zbuf, acc_sh, gsem, ssem,
             isem, zsem) = refs
        c = lax.axis_index("c")
        s = lax.axis_index("s")
        base = s * NCH
        stripe0 = s * STRIPE

        def _stage(g, slot):
            pltpu.async_copy(rc.at[0, pl.ds(base + g * G, G)],
                             idxr_v.at[slot], isem)
            pltpu.async_copy(rc.at[1, pl.ds(base + g * G, G)],
                             idxc_v.at[slot], isem)

        def _stage_wait(slot):
            pltpu.make_async_copy(rc.at[0, pl.ds(base, G)], idxr_v.at[slot],
                                  isem).wait()
            pltpu.make_async_copy(rc.at[1, pl.ds(base, G)], idxc_v.at[slot],
                                  isem).wait()

        def _zfill(i, carry):
            zbuf[i, pl.ds(0, 16)] = jnp.zeros((16,), _f32)
            zbuf[i, pl.ds(16, 16)] = jnp.zeros((16,), _f32)
            return carry

        lax.fori_loop(0, ZROWS, _zfill, 0)
        if with_cnt:
            for k in range(LANES // 16):
                ones_v[pl.ds(k * 16, 16)] = jnp.ones((16,), _f32)
            for k in range(ZROWS // 16):
                zrow[pl.ds(k * 16, 16)] = jnp.zeros((16,), _f32)

        _stage(0, 0)

        def _zacc(j, carry):
            pltpu.async_copy(zbuf, acc_sh.at[pl.ds(stripe0 + j * ZROWS, ZROWS)],
                             zsem)
            if with_cnt:
                pltpu.async_copy(zrow,
                                 cnt_sh.at[pl.ds(stripe0 + j * ZROWS, ZROWS)],
                                 zsem)
            return carry

        def _zdrain(j, carry):
            pltpu.make_async_copy(zbuf, acc_sh.at[pl.ds(stripe0, ZROWS)],
                                  zsem).wait()
            if with_cnt:
                pltpu.make_async_copy(zrow, cnt_sh.at[pl.ds(stripe0, ZROWS)],
                                      zsem).wait()
            return carry

        lax.fori_loop(0, STRIPE // ZROWS, _zacc, 0)
        lax.fori_loop(0, STRIPE // ZROWS, _zdrain, 0)
        plsc.subcore_barrier()

        def _gather(slot, j, buf):
            @pl.when(c == 0)
            def _():
                pltpu.async_copy(m_lo.at[idxr_v.at[slot, j]], val_v.at[buf],
                                 gsem[buf])

            @pl.when(c == 1)
            def _():
                pltpu.async_copy(m_hi.at[idxr_v.at[slot, j]], val_v.at[buf],
                                 gsem[buf])

        def _gwait(buf):
            pltpu.make_async_copy(m_lo.at[idxr_v.at[0, 0]], val_v.at[buf],
                                  gsem[buf]).wait()

        def _scatter(slot, j, buf):
            pltpu.async_copy(val_v.at[buf], acc_sh.at[idxc_v.at[slot, j]],
                             ssem[buf], add=True)

        def _swait(buf):
            pltpu.make_async_copy(val_v.at[buf], acc_sh.at[idxc_v.at[0, 0]],
                                  ssem[buf]).wait()

        def _cnt_active(g):
            return ((c == 0) & (g < NG // 2)) | ((c == 1) & (g >= NG // 2))

        def _count(g, slot, j):
            @pl.when(_cnt_active(g))
            def _():
                pltpu.async_copy(ones_v, cnt_sh.at[idxc_v.at[slot, j]], csem,
                                 add=True)

        def _group(g, carry):
            slot = jnp.bitwise_and(g, 1)
            _stage_wait(slot)

            @pl.when(g + 1 < NG)
            def _():
                _stage(g + 1, 1 - slot)

            for j in range(G):
                buf = j % R
                if j < R:
                    @pl.when(g > 0)
                    def _():
                        _swait(buf)
                else:
                    _swait(buf)
                _gather(slot, j, buf)
                if j >= 2:
                    _gwait((j - 2) % R)
                    _scatter(slot, j - 2, (j - 2) % R)
                    if with_cnt:
                        _count(g, slot, j - 2)
            for j in range(G - 2, G):
                _gwait(j % R)
                _scatter(slot, j, j % R)
                if with_cnt:
                    _count(g, slot, j)
            if with_cnt:
                @pl.when(_cnt_active(g))
                def _():
                    for _j in range(G):
                        pltpu.make_async_copy(
                            ones_v, cnt_sh.at[idxc_v.at[0, 0]], csem).wait()
            return carry

        lax.fori_loop(0, NG, _group, 0)
        for buf in range(R):
            _swait(buf)
        plsc.subcore_barrier()

        pltpu.sync_copy(acc_sh.at[pl.ds(stripe0, STRIPE)],
                        s_out.at[c, pl.ds(stripe0, STRIPE)])
        if with_cnt:
            pltpu.sync_copy(cnt_sh.at[pl.ds(stripe0, STRIPE)],
                            cnt_out.at[c, pl.ds(stripe0, STRIPE)])

    return pl.kernel(
        body, out_type=out_type, mesh=mesh, scratch_types=scratch,
        compiler_params=pltpu.CompilerParams(use_tc_tiling_on_sc=False))


def _dot(a, b):
    return jnp.dot(a, b, preferred_element_type=_f32)


def _t1_body(x_ref, wa, ba, wb, bb, lo_ref, hi_ref):
    h = jnp.maximum(_dot(x_ref[...], wa[...]) + ba[...], 0.0)
    m = _dot(h, wb[...]) + bb[...]
    lo_ref[...] = m[:, :HQ]
    hi_ref[...] = m[:, HQ:]


def _t2_body(x_ref, sp, c0, c1, w1c0, w1c1, b1c, w1d, b1d,
             w2a0, w2a1, b2a, w2b, b2b, u_ref, lo_ref, hi_ref):
    cnt = jnp.maximum(c0[...] + c1[...], 1.0).reshape(-1, 1)
    s = sp[...]
    agg = jnp.concatenate([s[0], s[1]], axis=1) / cnt
    t = _dot(x_ref[...], w1c0[...]) + _dot(agg, w1c1[...]) + b1c[...]
    u = _dot(jnp.maximum(t, 0.0), w1d[...]) + b1d[...]
    u_ref[...] = u
    t2 = _dot(x_ref[...], w2a0[...]) + _dot(u, w2a1[...]) + b2a[...]
    m2 = _dot(jnp.maximum(t2, 0.0), w2b[...]) + b2b[...]
    lo_ref[...] = m2[:, :HQ]
    hi_ref[...] = m2[:, HQ:]


def _t3_body(x_ref, u, sp, c0, c1, w2c0, w2c1, w2c2, b2c, wd, bd,
             o_ref):
    cnt = jnp.maximum(c0[...] + c1[...], 1.0).reshape(-1, 1)
    s = sp[...]
    agg = jnp.concatenate([s[0], s[1]], axis=1) / cnt
    t = (_dot(x_ref[...], w2c0[...]) + _dot(u[...], w2c1[...])
         + _dot(agg, w2c2[...]) + b2c[...])
    o_ref[...] = _dot(jnp.maximum(t, 0.0), wd[...]) + bd[...]


def _row_spec(width):
    if width is None:
        return pl.BlockSpec((BM,), lambda i: (i,))
    if width == "sp":
        return pl.BlockSpec((2, BM, HQ), lambda i: (0, i, 0))
    if width == "m":
        return pl.BlockSpec((BM, HQ), lambda i: (i, 0))
    return pl.BlockSpec((BM, width), lambda i: (i, 0))


def _full_spec(shape):
    return pl.BlockSpec(shape, lambda i: (0,) * len(shape))


def _tc_call(body, in_widths, full_shapes, out_widths):
    grid = (pl.cdiv(N, BM),)
    out_shape = [jax.ShapeDtypeStruct((ACC_ROWS, HQ) if w == "m" else (N, w),
                                      _f32) for w in out_widths]
    out_specs = [_row_spec(w) for w in out_widths]
    if len(out_widths) == 1:
        out_shape, out_specs = out_shape[0], out_specs[0]
    return pl.pallas_call(
        body,
        grid=grid,
        in_specs=[_row_spec(w) for w in in_widths] + [_full_spec(s) for s in full_shapes],
        out_specs=out_specs,
        out_shape=out_shape,
    )


def _edge_scatter(m_lo, m_hi, rc_p, first):
    if first:
        sa, cnt = _make_sc_pass(True)(m_lo, m_hi, rc_p)
        return sa, (cnt[0], cnt[1])
    (sa,) = _make_sc_pass(False)(m_lo, m_hi, rc_p)
    return sa, None


def kernel(x, edge_index, edge_attr, W1a, b1a, W1b, b1b, W1c, b1c, W1d, b1d,
           W2a, b2a, W2b, b2b, W2c, b2c, W2d, b2d):
    del edge_attr
    rc_p = jnp.pad(edge_index.astype(jnp.int32), ((0, 0), (0, EP - E)),
                   constant_values=TRASH).reshape(2, EROWS, LANES)

    r2 = lambda b: b.reshape(1, -1)

    m1_lo, m1_hi = _tc_call(
        _t1_body, [F_IN], [(F_IN, H), (1, H), (H, H), (1, H)], ["m", "m"])(
        x, W1a, r2(b1a), W1b, r2(b1b))
    s1p, (c0, c1) = _edge_scatter(m1_lo, m1_hi, rc_p, first=True)

    u, m2_lo, m2_hi = _tc_call(
        _t2_body, [F_IN, "sp", None, None],
        [(F_IN, H), (H, H), (1, H), (H, H), (1, H),
         (F_IN, H), (H, H), (1, H), (H, H), (1, H)],
        [H, "m", "m"],
    )(x, s1p, c0, c1,
      W1c[:F_IN], W1c[F_IN:], r2(b1c), W1d, r2(b1d),
      W2a[:F_IN], W2a[F_IN:], r2(b2a), W2b, r2(b2b))

    s2p, _ = _edge_scatter(m2_lo, m2_hi, rc_p, first=False)

    out = _tc_call(
        _t3_body, [F_IN, H, "sp", None, None],
        [(F_IN, H), (H, H), (H, H), (1, H), (H, OUT), (1, OUT)],
        [OUT],
    )(x, u, s2p, c0, c1,
      W2c[:F_IN], W2c[F_IN:F_IN + H], W2c[F_IN + H:], r2(b2c), W2d, r2(b2d))
    return out
